# Initial kernel scaffold; baseline (speedup 1.0000x reference)
import jax
import jax.numpy as jnp
from jax.experimental import pallas as pl


def kernel(x, rbf, sbf, idx_kj, idx_ji, idx_i, W_ji, b_ji, W_kj, b_kj, W_rbf1, W_rbf2, W_sbf1, W_sbf2, W_down, W_up, bW, bB, W_lin, b_lin, aW, aB, W_orbf, W_oup, b_oup, oW, oB, W_out):
    raise NotImplementedError("write your pallas kernel here")


# speedup vs baseline: 3.0159x; 3.0159x over previous
import functools

import jax
import jax.numpy as jnp
from jax import lax
from jax.experimental import pallas as pl
from jax.experimental.pallas import tpu as pltpu
from jax.experimental.pallas import tpu_sc as plsc

N_NODES = 10000
NC = 2
NS = 16
PRB = 8192
NBIN = 20
NBPC = 10


def _silu(v):
    return v * (1.0 / (1.0 + jnp.exp(-v)))


def _dot(a, b):
    return jnp.dot(a, b, preferred_element_type=jnp.float32)


def _full(a):
    return pl.BlockSpec(a.shape, lambda *args: (0,) * a.ndim)


def _k1(x, rbf, W_ji, b_ji, W_kj, b_kj, W_rbf1, W_rbf2, W_orbf):
    E, H = x.shape
    R = rbf.shape[1]
    BE = 640
    grid = (E // BE,)

    def body(x_r, rbf_r, Wji_r, bji_r, Wkj_r, bkj_r, Wr1_r, Wr2_r,
             Wo_r, xji_o, xkj_o, orbf_o):
        xb = x_r[...]
        rb = rbf_r[...]
        xji_o[...] = _silu(_dot(xb, Wji_r[...]) + bji_r[...])
        rbf_e = _dot(_dot(rb, Wr1_r[...]), Wr2_r[...])
        xkj_o[...] = _silu(_dot(xb, Wkj_r[...]) + bkj_r[...]) * rbf_e
        orbf_o[...] = _dot(rb, Wo_r[...])

    return pl.pallas_call(
        body,
        grid=grid,
        in_specs=[
            pl.BlockSpec((BE, H), lambda i: (i, 0)),
            pl.BlockSpec((BE, R), lambda i: (i, 0)),
            _full(W_ji), _full(b_ji), _full(W_kj), _full(b_kj),
            _full(W_rbf1), _full(W_rbf2), _full(W_orbf),
        ],
        out_specs=[
            pl.BlockSpec((BE, H), lambda i: (i, 0)),
            pl.BlockSpec((BE, H), lambda i: (i, 0)),
            pl.BlockSpec((BE, H), lambda i: (i, 0)),
        ],
        out_shape=[
            jax.ShapeDtypeStruct((E, H), jnp.float32),
            jax.ShapeDtypeStruct((E, H), jnp.float32),
            jax.ShapeDtypeStruct((E, H), jnp.float32),
        ],
    )(x, rbf, W_ji, b_ji, W_kj, b_kj, W_rbf1, W_rbf2, W_orbf)


def _k2_gather(table, idx_kj):
    E, H = table.shape
    T = idx_kj.shape[0]
    NW = NC * NS
    TPW = T // NW
    SB = 128
    NFULL = TPW // SB
    TAIL = TPW - NFULL * SB
    mesh = plsc.VectorSubcoreMesh(core_axis_name="c", subcore_axis_name="s")

    @functools.partial(
        pl.kernel,
        mesh=mesh,
        out_type=jax.ShapeDtypeStruct((T, H), jnp.float32),
        scratch_types=[
            pltpu.VMEM((NFULL + 1, SB), jnp.int32),
            pltpu.VMEM((SB, H), jnp.float32),
            pltpu.VMEM((TAIL, H), jnp.float32),
            pltpu.SemaphoreType.DMA,
        ],
    )
    def k(tab_hbm, idx_hbm, g_hbm, idx_v, rows_v, tail_v, sem):
        c = lax.axis_index("c")
        s = lax.axis_index("s")
        base = (s * NC + c) * TPW

        def load_idx(b, _):
            pltpu.sync_copy(idx_hbm.at[pl.ds(base + b * SB, SB)], idx_v.at[b])
            return 0
        lax.fori_loop(0, NFULL, load_idx, 0)
        pltpu.sync_copy(idx_hbm.at[pl.ds(base + NFULL * SB, TAIL)],
                        idx_v.at[NFULL, pl.ds(0, TAIL)])

        def gath(b, _):
            pltpu.async_copy(tab_hbm.at[idx_v.at[b]], rows_v, sem).wait()
            pltpu.sync_copy(rows_v, g_hbm.at[pl.ds(base + b * SB, SB)])
            return 0
        lax.fori_loop(0, NFULL, gath, 0)
        pltpu.async_copy(tab_hbm.at[idx_v.at[NFULL, pl.ds(0, TAIL)]],
                         tail_v, sem).wait()
        pltpu.sync_copy(tail_v, g_hbm.at[pl.ds(base + NFULL * SB, TAIL)])

    return k(table, idx_kj)


def _k3(sbf, g, W_sbf1, W_sbf2, W_down, W_up):
    T, SR = sbf.shape
    H = g.shape[1]
    BT = 1024
    grid = (T // BT,)

    def body(sbf_r, g_r, W1_r, W2_r, Wd_r, Wu_r, mu_o):
        z = _dot(_dot(sbf_r[...], W1_r[...]), W2_r[...])
        xkd = _silu(_dot(g_r[...], Wd_r[...]))
        mu_o[...] = _dot(xkd * z, Wu_r[...])

    return pl.pallas_call(
        body,
        grid=grid,
        in_specs=[
            pl.BlockSpec((BT, SR), lambda i: (i, 0)),
            pl.BlockSpec((BT, H), lambda i: (i, 0)),
            _full(W_sbf1), _full(W_sbf2), _full(W_down), _full(W_up),
        ],
        out_specs=pl.BlockSpec((BT, H), lambda i: (i, 0)),
        out_shape=jax.ShapeDtypeStruct((T, H), jnp.float32),
    )(sbf, g, W_sbf1, W_sbf2, W_down, W_up)


def _r2a(idx2):
    NR = idx2.shape[0]
    grid = (NR // 8,)

    def body(ix_r, pl_o, c_o):
        d = ix_r[...] >> 13
        rows = []
        for b in range(NBIN):
            rows.append(jnp.where(d == b, 1.0, 0.0))
        OS = jnp.concatenate(rows, axis=0)

        gi = lax.broadcasted_iota(jnp.int32, (NBIN * 8, NBIN * 8), 0)
        gj = lax.broadcasted_iota(jnp.int32, (NBIN * 8, NBIN * 8), 1)
        BD = jnp.where((gi // 8 == gj // 8) & (gj < gi), 1.0, 0.0)
        rowtot = _dot(OS, jnp.ones((128, 1), jnp.float32))
        RP = _dot(BD, rowtot)

        li = lax.broadcasted_iota(jnp.int32, (128, 128), 0)
        lj = lax.broadcasted_iota(jnp.int32, (128, 128), 1)
        U = jnp.where(li < lj, 1.0, 0.0)
        LP = _dot(OS, U)

        pos = jnp.zeros((8, 128), jnp.float32)
        cnt = jnp.zeros((1, 128), jnp.float32)
        for b in range(NBIN):
            Ob = OS[8 * b:8 * b + 8]
            contrib = Ob * (RP[8 * b:8 * b + 8] + LP[8 * b:8 * b + 8])
            pos = pos + contrib
            tot = jnp.sum(Ob)
            oh = jnp.where(
                lax.broadcasted_iota(jnp.int32, (1, 128), 1) == b, 1.0, 0.0)
            cnt = cnt + tot * oh
        pl_o[...] = pos.astype(jnp.int32)
        c_o[...] = cnt[:, :32].reshape(1, 1, 32)

    return pl.pallas_call(
        body,
        grid=grid,
        in_specs=[pl.BlockSpec((8, 128), lambda i: (i, 0))],
        out_specs=[
            pl.BlockSpec((8, 128), lambda i: (i, 0)),
            pl.BlockSpec((1, 1, 32), lambda i: (i, 0, 0)),
        ],
        out_shape=[
            jax.ShapeDtypeStruct((NR, 128), jnp.int32),
            jax.ShapeDtypeStruct((NR // 8, 1, 32), jnp.float32),
        ],
    )(idx2)


def _r2b(C):
    NB = C.shape[0]

    def body(c_r, bm_o, st_o):
        Cv = c_r[...].reshape(NB, 32)
        tot = jnp.sum(Cv, axis=0, keepdims=True)
        bi = lax.broadcasted_iota(jnp.int32, (32, 32), 0)
        bj = lax.broadcasted_iota(jnp.int32, (32, 32), 1)
        U32 = jnp.where(bi < bj, 1.0, 0.0)
        start = _dot(tot, U32)
        ri = lax.broadcasted_iota(jnp.int32, (NB, NB), 0)
        rj = lax.broadcasted_iota(jnp.int32, (NB, NB), 1)
        UB = jnp.where(rj < ri, 1.0, 0.0)
        blkpfx = _dot(UB, Cv)
        bm_o[...] = (blkpfx + start).reshape(NB, 1, 32)
        st_o[...] = start.astype(jnp.int32)

    return pl.pallas_call(
        body,
        in_specs=[_full(C)],
        out_specs=[
            pl.BlockSpec((NB, 1, 32), lambda: (0, 0, 0)),
            pl.BlockSpec((1, 32), lambda: (0, 0)),
        ],
        out_shape=[
            jax.ShapeDtypeStruct((NB, 1, 32), jnp.float32),
            jax.ShapeDtypeStruct((1, 32), jnp.int32),
        ],
    )(C)


def _r2c(idx2, pos_local, Bm):
    NR = idx2.shape[0]
    grid = (NR // 8,)

    def body(ix_r, pl_r, bm_r, pos_o):
        d = ix_r[...] >> 13
        pos = pl_r[...].astype(jnp.float32)
        for b in range(NBIN):
            Ob = jnp.where(d == b, 1.0, 0.0)
            pos = pos + Ob * bm_r[0, 0, b]
        pos_o[...] = pos.astype(jnp.int32)

    return pl.pallas_call(
        body,
        grid=grid,
        in_specs=[
            pl.BlockSpec((8, 128), lambda i: (i, 0)),
            pl.BlockSpec((8, 128), lambda i: (i, 0)),
            pl.BlockSpec((1, 1, 32), lambda i: (i, 0, 0)),
        ],
        out_specs=pl.BlockSpec((8, 128), lambda i: (i, 0)),
        out_shape=jax.ShapeDtypeStruct((NR, 128), jnp.int32),
    )(idx2, pos_local, Bm)


def _k4a(idx2, pos):
    NR = idx2.shape[0]
    T = NR * 128
    NW = NC * NS
    RPW = NR // NW
    REM = NR - RPW * NW
    mesh = plsc.VectorSubcoreMesh(core_axis_name="c", subcore_axis_name="s")

    @functools.partial(
        pl.kernel,
        mesh=mesh,
        out_type=[
            jax.ShapeDtypeStruct((T,), jnp.int32),
            jax.ShapeDtypeStruct((T,), jnp.int32),
        ],
        scratch_types=[
            pltpu.VMEM((1, 128), jnp.int32),
            pltpu.VMEM((1, 128), jnp.int32),
            pltpu.VMEM((1, 128), jnp.int32),
        ],
    )
    def k(ix_hbm, pos_hbm, tid_hbm, dst_hbm, posb, valb, tidb):
        c = lax.axis_index("c")
        s = lax.axis_index("s")
        w = s * NC + c

        def do_row(row, _):
            pltpu.sync_copy(pos_hbm.at[row], posb.at[0])
            pltpu.sync_copy(ix_hbm.at[row], valb.at[0])
            for j in range(8):
                tidb[0, pl.ds(16 * j, 16)] = (
                    row * 128 + 16 * j + lax.iota(jnp.int32, 16))
            pltpu.sync_copy(valb.at[0], dst_hbm.at[posb.at[0]])
            pltpu.sync_copy(tidb.at[0], tid_hbm.at[posb.at[0]])
            return 0

        def loop(i, _):
            do_row(w * RPW + i, 0)
            return 0
        lax.fori_loop(0, RPW, loop, 0)

        @pl.when(w < REM)
        def _():
            do_row(NW * RPW + w, 0)

    return k(idx2, pos)


def _k4b(mu, tid_sorted, dst_sorted, starti, zeros4):
    T, H = mu.shape
    ACCR = 8448
    SB = 64
    mesh = plsc.VectorSubcoreMesh(core_axis_name="c", subcore_axis_name="s")

    @functools.partial(
        pl.kernel,
        mesh=mesh,
        out_type=jax.ShapeDtypeStruct((NBIN * PRB, H), jnp.float32),
        scratch_types=[
            pltpu.VMEM((1, 32), jnp.int32),
            pltpu.VMEM((1, SB), jnp.int32),
            pltpu.VMEM((SB,), jnp.int32),
            pltpu.VMEM((1, SB), jnp.int32),
            pltpu.VMEM((SB, H), jnp.float32),
            pltpu.VMEM((16, H), jnp.float32),
            pltpu.VMEM_SHARED((ACCR, H), jnp.float32),
            pltpu.SemaphoreType.DMA,
        ],
    )
    def k(mu_hbm, tid_hbm, dst_hbm, st_hbm, z_hbm, seg_hbm,
          startv, tidb, db, drow, gbuf, zbuf, acc, sem):
        c = lax.axis_index("c")
        s = lax.axis_index("s")
        pltpu.sync_copy(z_hbm, zbuf)
        pltpu.sync_copy(st_hbm, startv)
        v0 = startv[0, pl.ds(0, 16)]
        v1 = startv[0, pl.ds(16, 16)]

        def get_start(kk):
            return v0[kk] if kk < 16 else v1[kk - 16]

        for p in range(NBPC):
            st_a, en_a = get_start(p), get_start(p + 1)
            st_b, en_b = get_start(10 + p), get_start(11 + p)
            st = st_a * (1 - c) + st_b * c
            en = en_a * (1 - c) + en_b * c
            st = jnp.minimum(jnp.maximum(st, 0), T)
            en = jnp.minimum(jnp.maximum(en, st), T)
            lo = (c * NBPC + p) * PRB

            def zero(j, _):
                pltpu.sync_copy(zbuf, acc.at[pl.ds(s * 528 + j * 16, 16)])
                return 0
            lax.fori_loop(0, 33, zero, 0)
            plsc.subcore_barrier()

            b0 = (st >> 6)
            nb = ((en + SB - 1) >> 6) - b0
            nloc = jnp.maximum(nb - s + NS - 1, 0) // NS

            def batch(i, _):
                t0 = (b0 + s + NS * i) * SB
                pltpu.sync_copy(tid_hbm.at[pl.ds(t0, SB)], tidb.at[0])
                for m in range(SB // 16):
                    tv = tidb[0, pl.ds(16 * m, 16)]
                    tidb[0, pl.ds(16 * m, 16)] = jnp.minimum(
                        jnp.maximum(tv, 0), T - 1)
                pltpu.async_copy(mu_hbm.at[tidb.at[0]], gbuf, sem).wait()
                pltpu.sync_copy(dst_hbm.at[pl.ds(t0, SB)], db)
                for m in range(SB // 16):
                    v = db[pl.ds(16 * m, 16)]
                    oc = jnp.minimum(jnp.maximum(v - lo, -8), PRB) + 8
                    drow[0, pl.ds(16 * m, 16)] = oc
                pltpu.sync_copy(gbuf, acc.at[drow.at[0]], add=True)
                return 0
            lax.fori_loop(0, nloc, batch, 0)
            plsc.subcore_barrier()

            pltpu.sync_copy(acc.at[pl.ds(8 + s * 512, 512)],
                            seg_hbm.at[pl.ds(lo + s * 512, 512)])
            plsc.subcore_barrier()

    return k(mu, tid_sorted, dst_sorted, starti, zeros4)


def _k5(seg, x_ji, x, orbf, bW, bB, W_lin, b_lin, aW, aB):
    E, H = x.shape
    BE = 640
    grid = (E // BE,)

    def body(seg_r, xji_r, x_r, orbf_r, bW_r, bB_r, Wl_r, bl_r,
             aW_r, aB_r, t_o):
        h = xji_r[...] + _silu(seg_r[...])
        for l in range(bW_r.shape[0]):
            u = _silu(_dot(h, bW_r[l, 0]) + bB_r[l, 0])
            h = h + _silu(_dot(u, bW_r[l, 1]) + bB_r[l, 1])
        h = _silu(_dot(h, Wl_r[...]) + bl_r[...]) + x_r[...]
        for l in range(aW_r.shape[0]):
            u = _silu(_dot(h, aW_r[l, 0]) + aB_r[l, 0])
            h = h + _silu(_dot(u, aW_r[l, 1]) + aB_r[l, 1])
        t_o[...] = orbf_r[...] * h

    return pl.pallas_call(
        body,
        grid=grid,
        in_specs=[
            pl.BlockSpec((BE, H), lambda i: (i, 0)),
            pl.BlockSpec((BE, H), lambda i: (i, 0)),
            pl.BlockSpec((BE, H), lambda i: (i, 0)),
            pl.BlockSpec((BE, H), lambda i: (i, 0)),
            _full(bW), _full(bB), _full(W_lin), _full(b_lin),
            _full(aW), _full(aB),
        ],
        out_specs=pl.BlockSpec((BE, H), lambda i: (i, 0)),
        out_shape=jax.ShapeDtypeStruct((E, H), jnp.float32),
    )(seg, x_ji, x, orbf, bW, bB, W_lin, b_lin, aW, aB)


def _k6_node_scatter(t_arr, idx_i, zeros6):
    E, H = t_arr.shape
    EPC = E // NC
    EPT = EPC // NS
    SB = 128
    NFULL = EPT // SB
    TAIL = EPT - NFULL * SB
    ACCR = 10240
    mesh = plsc.VectorSubcoreMesh(core_axis_name="c", subcore_axis_name="s")

    @functools.partial(
        pl.kernel,
        mesh=mesh,
        out_type=jax.ShapeDtypeStruct((NC, N_NODES, H), jnp.float32),
        scratch_types=[
            pltpu.VMEM((NFULL + 1, SB), jnp.int32),
            pltpu.VMEM((SB, H), jnp.float32),
            pltpu.VMEM((TAIL, H), jnp.float32),
            pltpu.VMEM((16, H), jnp.float32),
            pltpu.VMEM_SHARED((ACCR, H), jnp.float32),
            pltpu.SemaphoreType.DMA,
        ],
    )
    def k(t_hbm, idx_hbm, z_hbm, part_hbm, idx_v, mbuf, tbuf, zbuf, acc, sem):
        c = lax.axis_index("c")
        s = lax.axis_index("s")
        ebase = c * EPC + s * EPT

        def load_idx(b, _):
            pltpu.sync_copy(idx_hbm.at[pl.ds(ebase + b * SB, SB)], idx_v.at[b])
            return 0
        lax.fori_loop(0, NFULL, load_idx, 0)
        pltpu.sync_copy(idx_hbm.at[pl.ds(ebase + NFULL * SB, TAIL)],
                        idx_v.at[NFULL, pl.ds(0, TAIL)])

        pltpu.sync_copy(z_hbm, zbuf)

        def zero(j, _):
            pltpu.sync_copy(zbuf, acc.at[pl.ds(s * 640 + j * 16, 16)])
            return 0
        lax.fori_loop(0, 40, zero, 0)
        plsc.subcore_barrier()

        def scat(b, _):
            pltpu.sync_copy(t_hbm.at[pl.ds(ebase + b * SB, SB)], mbuf)
            pltpu.sync_copy(mbuf, acc.at[idx_v.at[b]], add=True)
            return 0
        lax.fori_loop(0, NFULL, scat, 0)
        pltpu.sync_copy(t_hbm.at[pl.ds(ebase + NFULL * SB, TAIL)], tbuf)
        pltpu.sync_copy(tbuf, acc.at[idx_v.at[NFULL, pl.ds(0, TAIL)]],
                        add=True)
        plsc.subcore_barrier()

        pltpu.sync_copy(acc.at[pl.ds(s * 624, 624)],
                        part_hbm.at[c, pl.ds(s * 624, 624)])

        @pl.when(s == NS - 1)
        def _():
            pltpu.sync_copy(acc.at[pl.ds(9984, 16)],
                            part_hbm.at[c, pl.ds(9984, 16)])

    return k(t_arr, idx_i, zeros6)


def _k7(part, W_oup, b_oup, oW, oB, W_out):
    H = part.shape[2]
    OC = W_out.shape[1]
    BN = 2000
    grid = (N_NODES // BN,)

    def body(p_r, Wo_r, bo_r, oW_r, oB_r, Wout_r, out_o):
        tt = p_r[0] + p_r[1]
        y = _dot(tt, Wo_r[...]) + bo_r[...]
        for l in range(oW_r.shape[0]):
            y = _silu(_dot(y, oW_r[l]) + oB_r[l])
        out_o[...] = _dot(y, Wout_r[...])

    return pl.pallas_call(
        body,
        grid=grid,
        in_specs=[
            pl.BlockSpec((NC, BN, H), lambda i: (0, i, 0)),
            _full(W_oup), _full(b_oup), _full(oW), _full(oB), _full(W_out),
        ],
        out_specs=pl.BlockSpec((BN, OC), lambda i: (i, 0)),
        out_shape=jax.ShapeDtypeStruct((N_NODES, OC), jnp.float32),
    )(part, W_oup, b_oup, oW, oB, W_out)


def kernel(x, rbf, sbf, idx_kj, idx_ji, idx_i,
           W_ji, b_ji, W_kj, b_kj, W_rbf1, W_rbf2, W_sbf1, W_sbf2,
           W_down, W_up, bW, bB, W_lin, b_lin, aW, aB,
           W_orbf, W_oup, b_oup, oW, oB, W_out):
    H = x.shape[1]
    T = idx_ji.shape[0]
    zeros16 = jnp.zeros((16, H), jnp.float32)

    x_ji, xkj_mid, orbf = _k1(x, rbf, W_ji, b_ji, W_kj, b_kj,
                              W_rbf1, W_rbf2, W_orbf)
    g = _k2_gather(xkj_mid, idx_kj)
    mu = _k3(sbf, g, W_sbf1, W_sbf2, W_down, W_up)

    idx2 = idx_ji.reshape(T // 128, 128)
    pos_local, C = _r2a(idx2)
    Bm, starti = _r2b(C)
    pos = _r2c(idx2, pos_local, Bm)
    tid_sorted, dst_sorted = _k4a(idx2, pos)
    seg_pad = _k4b(mu, tid_sorted, dst_sorted, starti, zeros16)

    t_arr = _k5(seg_pad, x_ji, x, orbf, bW, bB, W_lin, b_lin, aW, aB)
    part = _k6_node_scatter(t_arr, idx_i, zeros16)
    return _k7(part, W_oup, b_oup, oW, oB, W_out)


# speedup vs baseline: 4.0050x; 1.3280x over previous
import functools

import jax
import jax.numpy as jnp
from jax import lax
from jax.experimental import pallas as pl
from jax.experimental.pallas import tpu as pltpu
from jax.experimental.pallas import tpu_sc as plsc

N_NODES = 10000
NC = 2
NS = 16
PRB = 8192
NBIN = 20
NBPC = 10


def _silu(v):
    return v * (1.0 / (1.0 + jnp.exp(-v)))


def _dot(a, b):
    return jnp.dot(a, b, preferred_element_type=jnp.float32)


def _full(a):
    return pl.BlockSpec(a.shape, lambda *args: (0,) * a.ndim)


def _k1(x, rbf, W_ji, b_ji, W_kj, b_kj, W_rbf1, W_rbf2, W_orbf):
    E, H = x.shape
    R = rbf.shape[1]
    BE = 640
    grid = (E // BE,)

    def body(x_r, rbf_r, Wji_r, bji_r, Wkj_r, bkj_r, Wr1_r, Wr2_r,
             Wo_r, xji_o, xkj_o, orbf_o):
        xb = x_r[...]
        rb = rbf_r[...]
        xji_o[...] = _silu(_dot(xb, Wji_r[...]) + bji_r[...])
        rbf_e = _dot(_dot(rb, Wr1_r[...]), Wr2_r[...])
        xkj_o[...] = _silu(_dot(xb, Wkj_r[...]) + bkj_r[...]) * rbf_e
        orbf_o[...] = _dot(rb, Wo_r[...])

    return pl.pallas_call(
        body,
        grid=grid,
        in_specs=[
            pl.BlockSpec((BE, H), lambda i: (i, 0)),
            pl.BlockSpec((BE, R), lambda i: (i, 0)),
            _full(W_ji), _full(b_ji), _full(W_kj), _full(b_kj),
            _full(W_rbf1), _full(W_rbf2), _full(W_orbf),
        ],
        out_specs=[
            pl.BlockSpec((BE, H), lambda i: (i, 0)),
            pl.BlockSpec((BE, H), lambda i: (i, 0)),
            pl.BlockSpec((BE, H), lambda i: (i, 0)),
        ],
        out_shape=[
            jax.ShapeDtypeStruct((E, H), jnp.float32),
            jax.ShapeDtypeStruct((E, H), jnp.float32),
            jax.ShapeDtypeStruct((E, H), jnp.float32),
        ],
    )(x, rbf, W_ji, b_ji, W_kj, b_kj, W_rbf1, W_rbf2, W_orbf)


def _k2_gather(table, idx_kj):
    E, H = table.shape
    T = idx_kj.shape[0]
    NW = NC * NS
    TPW = T // NW
    SB = 128
    NFULL = TPW // SB
    TAIL = TPW - NFULL * SB
    mesh = plsc.VectorSubcoreMesh(core_axis_name="c", subcore_axis_name="s")

    @functools.partial(
        pl.kernel,
        mesh=mesh,
        out_type=jax.ShapeDtypeStruct((T, H), jnp.float32),
        scratch_types=[
            pltpu.VMEM((NFULL + 1, SB), jnp.int32),
            pltpu.VMEM((SB, H), jnp.float32),
            pltpu.VMEM((TAIL, H), jnp.float32),
            pltpu.SemaphoreType.DMA,
        ],
    )
    def k(tab_hbm, idx_hbm, g_hbm, idx_v, rows_v, tail_v, sem):
        c = lax.axis_index("c")
        s = lax.axis_index("s")
        base = (s * NC + c) * TPW

        def load_idx(b, _):
            pltpu.sync_copy(idx_hbm.at[pl.ds(base + b * SB, SB)], idx_v.at[b])
            return 0
        lax.fori_loop(0, NFULL, load_idx, 0)
        pltpu.sync_copy(idx_hbm.at[pl.ds(base + NFULL * SB, TAIL)],
                        idx_v.at[NFULL, pl.ds(0, TAIL)])

        def gath(b, _):
            pltpu.async_copy(tab_hbm.at[idx_v.at[b]], rows_v, sem).wait()
            pltpu.sync_copy(rows_v, g_hbm.at[pl.ds(base + b * SB, SB)])
            return 0
        lax.fori_loop(0, NFULL, gath, 0)
        pltpu.async_copy(tab_hbm.at[idx_v.at[NFULL, pl.ds(0, TAIL)]],
                         tail_v, sem).wait()
        pltpu.sync_copy(tail_v, g_hbm.at[pl.ds(base + NFULL * SB, TAIL)])

    return k(table, idx_kj)


def _k3(sbf, g, W_sbf1, W_sbf2, W_down, W_up):
    T, SR = sbf.shape
    H = g.shape[1]
    BT = 1024
    grid = (T // BT,)

    def body(sbf_r, g_r, W1_r, W2_r, Wd_r, Wu_r, mu_o):
        z = _dot(_dot(sbf_r[...], W1_r[...]), W2_r[...])
        xkd = _silu(_dot(g_r[...], Wd_r[...]))
        mu_o[...] = _dot(xkd * z, Wu_r[...])

    return pl.pallas_call(
        body,
        grid=grid,
        in_specs=[
            pl.BlockSpec((BT, SR), lambda i: (i, 0)),
            pl.BlockSpec((BT, H), lambda i: (i, 0)),
            _full(W_sbf1), _full(W_sbf2), _full(W_down), _full(W_up),
        ],
        out_specs=pl.BlockSpec((BT, H), lambda i: (i, 0)),
        out_shape=jax.ShapeDtypeStruct((T, H), jnp.float32),
    )(sbf, g, W_sbf1, W_sbf2, W_down, W_up)


def _r2a(idx2):
    NR = idx2.shape[0]
    grid = (NR // 8,)

    def body(ix_r, pl_o, c_o):
        d = ix_r[...] >> 13
        rows = []
        for b in range(NBIN):
            rows.append(jnp.where(d == b, 1.0, 0.0))
        OS = jnp.concatenate(rows, axis=0)

        gi = lax.broadcasted_iota(jnp.int32, (NBIN * 8, NBIN * 8), 0)
        gj = lax.broadcasted_iota(jnp.int32, (NBIN * 8, NBIN * 8), 1)
        BD = jnp.where((gi // 8 == gj // 8) & (gj < gi), 1.0, 0.0)
        rowtot = _dot(OS, jnp.ones((128, 1), jnp.float32))
        RP = _dot(BD, rowtot)

        li = lax.broadcasted_iota(jnp.int32, (128, 128), 0)
        lj = lax.broadcasted_iota(jnp.int32, (128, 128), 1)
        U = jnp.where(li < lj, 1.0, 0.0)
        LP = _dot(OS, U)

        pos = jnp.zeros((8, 128), jnp.float32)
        cnt = jnp.zeros((1, 128), jnp.float32)
        for b in range(NBIN):
            Ob = OS[8 * b:8 * b + 8]
            contrib = Ob * (RP[8 * b:8 * b + 8] + LP[8 * b:8 * b + 8])
            pos = pos + contrib
            tot = jnp.sum(Ob)
            oh = jnp.where(
                lax.broadcasted_iota(jnp.int32, (1, 128), 1) == b, 1.0, 0.0)
            cnt = cnt + tot * oh
        pl_o[...] = pos.astype(jnp.int32)
        c_o[...] = cnt[:, :32].reshape(1, 1, 32)

    return pl.pallas_call(
        body,
        grid=grid,
        in_specs=[pl.BlockSpec((8, 128), lambda i: (i, 0))],
        out_specs=[
            pl.BlockSpec((8, 128), lambda i: (i, 0)),
            pl.BlockSpec((1, 1, 32), lambda i: (i, 0, 0)),
        ],
        out_shape=[
            jax.ShapeDtypeStruct((NR, 128), jnp.int32),
            jax.ShapeDtypeStruct((NR // 8, 1, 32), jnp.float32),
        ],
    )(idx2)


def _r2b(C):
    NB = C.shape[0]

    def body(c_r, bm_o, st_o):
        Cv = c_r[...].reshape(NB, 32)
        tot = jnp.sum(Cv, axis=0, keepdims=True)
        bi = lax.broadcasted_iota(jnp.int32, (32, 32), 0)
        bj = lax.broadcasted_iota(jnp.int32, (32, 32), 1)
        U32 = jnp.where(bi < bj, 1.0, 0.0)
        start = _dot(tot, U32)
        ri = lax.broadcasted_iota(jnp.int32, (NB, NB), 0)
        rj = lax.broadcasted_iota(jnp.int32, (NB, NB), 1)
        UB = jnp.where(rj < ri, 1.0, 0.0)
        blkpfx = _dot(UB, Cv)
        bm_o[...] = (blkpfx + start).reshape(NB, 1, 32)
        st_o[...] = start.astype(jnp.int32)

    return pl.pallas_call(
        body,
        in_specs=[_full(C)],
        out_specs=[
            pl.BlockSpec((NB, 1, 32), lambda: (0, 0, 0)),
            pl.BlockSpec((1, 32), lambda: (0, 0)),
        ],
        out_shape=[
            jax.ShapeDtypeStruct((NB, 1, 32), jnp.float32),
            jax.ShapeDtypeStruct((1, 32), jnp.int32),
        ],
    )(C)


def _r2c(idx2, pos_local, Bm):
    NR = idx2.shape[0]
    grid = (NR // 8,)

    def body(ix_r, pl_r, bm_r, pos_o):
        d = ix_r[...] >> 13
        pos = pl_r[...].astype(jnp.float32)
        for b in range(NBIN):
            Ob = jnp.where(d == b, 1.0, 0.0)
            pos = pos + Ob * bm_r[0, 0, b]
        pos_o[...] = pos.astype(jnp.int32)

    return pl.pallas_call(
        body,
        grid=grid,
        in_specs=[
            pl.BlockSpec((8, 128), lambda i: (i, 0)),
            pl.BlockSpec((8, 128), lambda i: (i, 0)),
            pl.BlockSpec((1, 1, 32), lambda i: (i, 0, 0)),
        ],
        out_specs=pl.BlockSpec((8, 128), lambda i: (i, 0)),
        out_shape=jax.ShapeDtypeStruct((NR, 128), jnp.int32),
    )(idx2, pos_local, Bm)


def _k4a(idx2, pos, zeros_i):
    NR = idx2.shape[0]
    T = NR * 128
    NW = NC * NS
    RPW = NR // NW
    REM = NR - RPW * NW
    WPT = T // NS
    ZB = 4000
    mesh = plsc.VectorSubcoreMesh(core_axis_name="c", subcore_axis_name="s")

    @functools.partial(
        pl.kernel,
        mesh=mesh,
        out_type=[
            jax.ShapeDtypeStruct((NC * T,), jnp.int32),
            jax.ShapeDtypeStruct((NC * T,), jnp.int32),
        ],
        scratch_types=[
            pltpu.VMEM((1, 128), jnp.int32),
            pltpu.VMEM((1, 128), jnp.int32),
            pltpu.VMEM((1, 128), jnp.int32),
            pltpu.VMEM((ZB,), jnp.int32),
            pltpu.VMEM((ZB,), jnp.int32),
            pltpu.VMEM_SHARED((T,), jnp.int32),
            pltpu.VMEM_SHARED((T,), jnp.int32),
        ],
    )
    def k(ix_hbm, pos_hbm, z_hbm, tid_hbm, dst_hbm,
          posb, valb, tidb, zb, sbuf, tidS, dstS):
        c = lax.axis_index("c")
        s = lax.axis_index("s")
        w = s * NC + c

        pltpu.sync_copy(z_hbm, zb)

        def zero(j, _):
            pltpu.sync_copy(zb, tidS.at[pl.ds(s * WPT + j * ZB, ZB)])
            pltpu.sync_copy(zb, dstS.at[pl.ds(s * WPT + j * ZB, ZB)])
            return 0
        lax.fori_loop(0, WPT // ZB, zero, 0)
        plsc.subcore_barrier()

        def do_row(row, _):
            pltpu.sync_copy(pos_hbm.at[row], posb.at[0])
            pltpu.sync_copy(ix_hbm.at[row], valb.at[0])
            for j in range(8):
                tidb[0, pl.ds(16 * j, 16)] = (
                    row * 128 + 16 * j + lax.iota(jnp.int32, 16))
            pltpu.sync_copy(valb.at[0], dstS.at[posb.at[0]], add=True)
            pltpu.sync_copy(tidb.at[0], tidS.at[posb.at[0]], add=True)
            return 0

        def loop(i, _):
            do_row(w * RPW + i, 0)
            return 0
        lax.fori_loop(0, RPW, loop, 0)

        @pl.when(w < REM)
        def _():
            do_row(NW * RPW + w, 0)

        plsc.subcore_barrier()

        def dump(j, _):
            off = s * WPT + j * ZB
            pltpu.sync_copy(tidS.at[pl.ds(off, ZB)], sbuf)
            pltpu.sync_copy(sbuf, tid_hbm.at[pl.ds(c * T + off, ZB)])
            pltpu.sync_copy(dstS.at[pl.ds(off, ZB)], sbuf)
            pltpu.sync_copy(sbuf, dst_hbm.at[pl.ds(c * T + off, ZB)])
            return 0
        lax.fori_loop(0, WPT // ZB, dump, 0)

    return k(idx2, pos, zeros_i)


def _k4b(mu, tidP, dstP, starti, zeros4):
    T = mu.shape[0]
    H = mu.shape[1]
    ACCR = 8448
    SB = 128
    mesh = plsc.VectorSubcoreMesh(core_axis_name="c", subcore_axis_name="s")

    @functools.partial(
        pl.kernel,
        mesh=mesh,
        out_type=jax.ShapeDtypeStruct((NBIN * PRB, H), jnp.float32),
        scratch_types=[
            pltpu.VMEM((1, 32), jnp.int32),
            pltpu.VMEM((1, SB), jnp.int32),
            pltpu.VMEM((1, SB), jnp.int32),
            pltpu.VMEM((SB,), jnp.int32),
            pltpu.VMEM((SB,), jnp.int32),
            pltpu.VMEM((1, SB), jnp.int32),
            pltpu.VMEM((SB, H), jnp.float32),
            pltpu.VMEM((16, H), jnp.float32),
            pltpu.VMEM_SHARED((ACCR, H), jnp.float32),
            pltpu.SemaphoreType.DMA,
        ],
    )
    def k(mu_hbm, tid_hbm, dst_hbm, st_hbm, z_hbm, seg_hbm,
          startv, tidb, tidb2, db, db2, drow, gbuf, zbuf, acc, sem):
        c = lax.axis_index("c")
        s = lax.axis_index("s")
        pltpu.sync_copy(z_hbm, zbuf)
        pltpu.sync_copy(st_hbm, startv)
        v0 = startv[0, pl.ds(0, 16)]
        v1 = startv[0, pl.ds(16, 16)]

        def get_start(kk):
            return v0[kk] if kk < 16 else v1[kk - 16]

        for p in range(NBPC):
            st_a, en_a = get_start(p), get_start(p + 1)
            st_b, en_b = get_start(10 + p), get_start(11 + p)
            st = st_a * (1 - c) + st_b * c
            en = en_a * (1 - c) + en_b * c
            st = jnp.minimum(jnp.maximum(st, 0), T)
            en = jnp.minimum(jnp.maximum(en, st), T)
            lo = (c * NBPC + p) * PRB

            def zero(j, _):
                pltpu.sync_copy(zbuf, acc.at[pl.ds(s * 528 + j * 16, 16)])
                return 0
            lax.fori_loop(0, 33, zero, 0)
            plsc.subcore_barrier()

            b0 = (st >> 7)
            nb = ((en + SB - 1) >> 7) - b0
            nloc = jnp.maximum(nb - s + NS - 1, 0) // NS

            def batch(i, _):
                t0 = (b0 + s + NS * i) * SB
                pltpu.sync_copy(tid_hbm.at[pl.ds(t0, SB)], tidb.at[0])
                pltpu.sync_copy(tid_hbm.at[pl.ds(T + t0, SB)], tidb2.at[0])
                for m in range(SB // 16):
                    tv = (tidb[0, pl.ds(16 * m, 16)]
                          + tidb2[0, pl.ds(16 * m, 16)])
                    tidb[0, pl.ds(16 * m, 16)] = jnp.minimum(
                        jnp.maximum(tv, 0), T - 1)
                pltpu.async_copy(mu_hbm.at[tidb.at[0]], gbuf, sem).wait()
                pltpu.sync_copy(dst_hbm.at[pl.ds(t0, SB)], db)
                pltpu.sync_copy(dst_hbm.at[pl.ds(T + t0, SB)], db2)
                for m in range(SB // 16):
                    v = db[pl.ds(16 * m, 16)] + db2[pl.ds(16 * m, 16)]
                    oc = jnp.minimum(jnp.maximum(v - lo, -8), PRB) + 8
                    drow[0, pl.ds(16 * m, 16)] = oc
                pltpu.sync_copy(gbuf, acc.at[drow.at[0]], add=True)
                return 0
            lax.fori_loop(0, nloc, batch, 0)
            plsc.subcore_barrier()

            pltpu.sync_copy(acc.at[pl.ds(8 + s * 512, 512)],
                            seg_hbm.at[pl.ds(lo + s * 512, 512)])
            plsc.subcore_barrier()

    return k(mu, tidP, dstP, starti, zeros4)


def _k5(seg, x_ji, x, orbf, bW, bB, W_lin, b_lin, aW, aB):
    E, H = x.shape
    BE = 640
    grid = (E // BE,)

    def body(seg_r, xji_r, x_r, orbf_r, bW_r, bB_r, Wl_r, bl_r,
             aW_r, aB_r, t_o):
        h = xji_r[...] + _silu(seg_r[...])
        for l in range(bW_r.shape[0]):
            u = _silu(_dot(h, bW_r[l, 0]) + bB_r[l, 0])
            h = h + _silu(_dot(u, bW_r[l, 1]) + bB_r[l, 1])
        h = _silu(_dot(h, Wl_r[...]) + bl_r[...]) + x_r[...]
        for l in range(aW_r.shape[0]):
            u = _silu(_dot(h, aW_r[l, 0]) + aB_r[l, 0])
            h = h + _silu(_dot(u, aW_r[l, 1]) + aB_r[l, 1])
        t_o[...] = orbf_r[...] * h

    return pl.pallas_call(
        body,
        grid=grid,
        in_specs=[
            pl.BlockSpec((BE, H), lambda i: (i, 0)),
            pl.BlockSpec((BE, H), lambda i: (i, 0)),
            pl.BlockSpec((BE, H), lambda i: (i, 0)),
            pl.BlockSpec((BE, H), lambda i: (i, 0)),
            _full(bW), _full(bB), _full(W_lin), _full(b_lin),
            _full(aW), _full(aB),
        ],
        out_specs=pl.BlockSpec((BE, H), lambda i: (i, 0)),
        out_shape=jax.ShapeDtypeStruct((E, H), jnp.float32),
    )(seg, x_ji, x, orbf, bW, bB, W_lin, b_lin, aW, aB)


def _k6_node_scatter(t_arr, idx_i, zeros6):
    E, H = t_arr.shape
    EPC = E // NC
    EPT = EPC // NS
    SB = 128
    NFULL = EPT // SB
    TAIL = EPT - NFULL * SB
    ACCR = 10240
    mesh = plsc.VectorSubcoreMesh(core_axis_name="c", subcore_axis_name="s")

    @functools.partial(
        pl.kernel,
        mesh=mesh,
        out_type=jax.ShapeDtypeStruct((NC, N_NODES, H), jnp.float32),
        scratch_types=[
            pltpu.VMEM((NFULL + 1, SB), jnp.int32),
            pltpu.VMEM((SB, H), jnp.float32),
            pltpu.VMEM((TAIL, H), jnp.float32),
            pltpu.VMEM((16, H), jnp.float32),
            pltpu.VMEM_SHARED((ACCR, H), jnp.float32),
            pltpu.SemaphoreType.DMA,
        ],
    )
    def k(t_hbm, idx_hbm, z_hbm, part_hbm, idx_v, mbuf, tbuf, zbuf, acc, sem):
        c = lax.axis_index("c")
        s = lax.axis_index("s")
        ebase = c * EPC + s * EPT

        def load_idx(b, _):
            pltpu.sync_copy(idx_hbm.at[pl.ds(ebase + b * SB, SB)], idx_v.at[b])
            return 0
        lax.fori_loop(0, NFULL, load_idx, 0)
        pltpu.sync_copy(idx_hbm.at[pl.ds(ebase + NFULL * SB, TAIL)],
                        idx_v.at[NFULL, pl.ds(0, TAIL)])

        pltpu.sync_copy(z_hbm, zbuf)

        def zero(j, _):
            pltpu.sync_copy(zbuf, acc.at[pl.ds(s * 640 + j * 16, 16)])
            return 0
        lax.fori_loop(0, 40, zero, 0)
        plsc.subcore_barrier()

        def scat(b, _):
            pltpu.sync_copy(t_hbm.at[pl.ds(ebase + b * SB, SB)], mbuf)
            pltpu.sync_copy(mbuf, acc.at[idx_v.at[b]], add=True)
            return 0
        lax.fori_loop(0, NFULL, scat, 0)
        pltpu.sync_copy(t_hbm.at[pl.ds(ebase + NFULL * SB, TAIL)], tbuf)
        pltpu.sync_copy(tbuf, acc.at[idx_v.at[NFULL, pl.ds(0, TAIL)]],
                        add=True)
        plsc.subcore_barrier()

        pltpu.sync_copy(acc.at[pl.ds(s * 624, 624)],
                        part_hbm.at[c, pl.ds(s * 624, 624)])

        @pl.when(s == NS - 1)
        def _():
            pltpu.sync_copy(acc.at[pl.ds(9984, 16)],
                            part_hbm.at[c, pl.ds(9984, 16)])

    return k(t_arr, idx_i, zeros6)


def _k7(part, W_oup, b_oup, oW, oB, W_out):
    H = part.shape[2]
    OC = W_out.shape[1]
    BN = 2000
    grid = (N_NODES // BN,)

    def body(p_r, Wo_r, bo_r, oW_r, oB_r, Wout_r, out_o):
        tt = p_r[0] + p_r[1]
        y = _dot(tt, Wo_r[...]) + bo_r[...]
        for l in range(oW_r.shape[0]):
            y = _silu(_dot(y, oW_r[l]) + oB_r[l])
        out_o[...] = _dot(y, Wout_r[...])

    return pl.pallas_call(
        body,
        grid=grid,
        in_specs=[
            pl.BlockSpec((NC, BN, H), lambda i: (0, i, 0)),
            _full(W_oup), _full(b_oup), _full(oW), _full(oB), _full(W_out),
        ],
        out_specs=pl.BlockSpec((BN, OC), lambda i: (i, 0)),
        out_shape=jax.ShapeDtypeStruct((N_NODES, OC), jnp.float32),
    )(part, W_oup, b_oup, oW, oB, W_out)


def kernel(x, rbf, sbf, idx_kj, idx_ji, idx_i,
           W_ji, b_ji, W_kj, b_kj, W_rbf1, W_rbf2, W_sbf1, W_sbf2,
           W_down, W_up, bW, bB, W_lin, b_lin, aW, aB,
           W_orbf, W_oup, b_oup, oW, oB, W_out):
    H = x.shape[1]
    T = idx_ji.shape[0]
    zeros16 = jnp.zeros((16, H), jnp.float32)

    x_ji, xkj_mid, orbf = _k1(x, rbf, W_ji, b_ji, W_kj, b_kj,
                              W_rbf1, W_rbf2, W_orbf)
    g = _k2_gather(xkj_mid, idx_kj)
    mu = _k3(sbf, g, W_sbf1, W_sbf2, W_down, W_up)

    idx2 = idx_ji.reshape(T // 128, 128)
    pos_local, C = _r2a(idx2)
    Bm, starti = _r2b(C)
    pos = _r2c(idx2, pos_local, Bm)
    zeros_i = jnp.zeros((4000,), jnp.int32)
    tidP, dstP = _k4a(idx2, pos, zeros_i)
    seg_pad = _k4b(mu, tidP, dstP, starti, zeros16)

    t_arr = _k5(seg_pad, x_ji, x, orbf, bW, bB, W_lin, b_lin, aW, aB)
    part = _k6_node_scatter(t_arr, idx_i, zeros16)
    return _k7(part, W_oup, b_oup, oW, oB, W_out)


# speedup vs baseline: 4.3548x; 1.0874x over previous
import functools

import jax
import jax.numpy as jnp
from jax import lax
from jax.experimental import pallas as pl
from jax.experimental.pallas import tpu as pltpu
from jax.experimental.pallas import tpu_sc as plsc

N_NODES = 10000
NC = 2
NS = 16
PRB = 8192
NBIN = 20
NBPC = 10


def _silu(v):
    return v * (1.0 / (1.0 + jnp.exp(-v)))


def _dot(a, b):
    return jnp.dot(a, b, preferred_element_type=jnp.float32)


def _full(a):
    return pl.BlockSpec(a.shape, lambda *args: (0,) * a.ndim)


def _k1(x, rbf, W_ji, b_ji, W_kj, b_kj, W_rbf1, W_rbf2, W_orbf):
    E, H = x.shape
    R = rbf.shape[1]
    BE = 640
    grid = (E // BE,)

    def body(x_r, rbf_r, Wji_r, bji_r, Wkj_r, bkj_r, Wr1_r, Wr2_r,
             Wo_r, xji_o, xkj_o, orbf_o):
        xb = x_r[...]
        rb = rbf_r[...]
        xji_o[...] = _silu(_dot(xb, Wji_r[...]) + bji_r[...])
        rbf_e = _dot(_dot(rb, Wr1_r[...]), Wr2_r[...])
        xkj_o[...] = _silu(_dot(xb, Wkj_r[...]) + bkj_r[...]) * rbf_e
        orbf_o[...] = _dot(rb, Wo_r[...])

    return pl.pallas_call(
        body,
        grid=grid,
        in_specs=[
            pl.BlockSpec((BE, H), lambda i: (i, 0)),
            pl.BlockSpec((BE, R), lambda i: (i, 0)),
            _full(W_ji), _full(b_ji), _full(W_kj), _full(b_kj),
            _full(W_rbf1), _full(W_rbf2), _full(W_orbf),
        ],
        out_specs=[
            pl.BlockSpec((BE, H), lambda i: (i, 0)),
            pl.BlockSpec((BE, H), lambda i: (i, 0)),
            pl.BlockSpec((BE, H), lambda i: (i, 0)),
        ],
        out_shape=[
            jax.ShapeDtypeStruct((E, H), jnp.float32),
            jax.ShapeDtypeStruct((E, H), jnp.float32),
            jax.ShapeDtypeStruct((E, H), jnp.float32),
        ],
    )(x, rbf, W_ji, b_ji, W_kj, b_kj, W_rbf1, W_rbf2, W_orbf)


def _k2_gather(table, idx_kj):
    E, H = table.shape
    T = idx_kj.shape[0]
    NW = NC * NS
    TPW = T // NW
    SB = 128
    NFULL = TPW // SB
    TAIL = TPW - NFULL * SB
    mesh = plsc.VectorSubcoreMesh(core_axis_name="c", subcore_axis_name="s")

    @functools.partial(
        pl.kernel,
        mesh=mesh,
        out_type=jax.ShapeDtypeStruct((T, H), jnp.float32),
        scratch_types=[
            pltpu.VMEM((NFULL + 1, SB), jnp.int32),
            pltpu.VMEM((SB, H), jnp.float32),
            pltpu.VMEM((SB, H), jnp.float32),
            pltpu.VMEM((TAIL, H), jnp.float32),
            pltpu.SemaphoreType.DMA,
            pltpu.SemaphoreType.DMA,
        ],
    )
    def k(tab_hbm, idx_hbm, g_hbm, idx_v, rows_v, rows2_v, tail_v, sem,
          sem2):
        c = lax.axis_index("c")
        s = lax.axis_index("s")
        base = (s * NC + c) * TPW

        def load_idx(b, _):
            pltpu.sync_copy(idx_hbm.at[pl.ds(base + b * SB, SB)], idx_v.at[b])
            return 0
        lax.fori_loop(0, NFULL, load_idx, 0)
        pltpu.sync_copy(idx_hbm.at[pl.ds(base + NFULL * SB, TAIL)],
                        idx_v.at[NFULL, pl.ds(0, TAIL)])

        def gath2(q, _):
            b0, b1 = 2 * q, 2 * q + 1
            d0 = pltpu.async_copy(tab_hbm.at[idx_v.at[b0]], rows_v, sem)
            d1 = pltpu.async_copy(tab_hbm.at[idx_v.at[b1]], rows2_v, sem2)
            d0.wait()
            pltpu.sync_copy(rows_v, g_hbm.at[pl.ds(base + b0 * SB, SB)])
            d1.wait()
            pltpu.sync_copy(rows2_v, g_hbm.at[pl.ds(base + b1 * SB, SB)])
            return 0
        lax.fori_loop(0, NFULL // 2, gath2, 0)
        pltpu.async_copy(tab_hbm.at[idx_v.at[NFULL, pl.ds(0, TAIL)]],
                         tail_v, sem).wait()
        pltpu.sync_copy(tail_v, g_hbm.at[pl.ds(base + NFULL * SB, TAIL)])

    return k(table, idx_kj)


def _k3(sbf, g, W_sbf1, W_sbf2, W_down, W_up):
    T, SR = sbf.shape
    H = g.shape[1]
    BT = 1024
    grid = (T // BT,)

    def body(sbf_r, g_r, W1_r, W2_r, Wd_r, Wu_r, mu_o):
        z = _dot(_dot(sbf_r[...], W1_r[...]), W2_r[...])
        xkd = _silu(_dot(g_r[...], Wd_r[...]))
        mu_o[...] = _dot(xkd * z, Wu_r[...])

    return pl.pallas_call(
        body,
        grid=grid,
        in_specs=[
            pl.BlockSpec((BT, SR), lambda i: (i, 0)),
            pl.BlockSpec((BT, H), lambda i: (i, 0)),
            _full(W_sbf1), _full(W_sbf2), _full(W_down), _full(W_up),
        ],
        out_specs=pl.BlockSpec((BT, H), lambda i: (i, 0)),
        out_shape=jax.ShapeDtypeStruct((T, H), jnp.float32),
    )(sbf, g, W_sbf1, W_sbf2, W_down, W_up)


def _r2a(idx2):
    NR = idx2.shape[0]
    grid = (NR // 8,)

    def body(ix_r, pl_o, c_o):
        d = ix_r[...] >> 13
        rows = []
        for b in range(NBIN):
            rows.append(jnp.where(d == b, 1.0, 0.0))
        OS = jnp.concatenate(rows, axis=0)

        gi = lax.broadcasted_iota(jnp.int32, (NBIN * 8, NBIN * 8), 0)
        gj = lax.broadcasted_iota(jnp.int32, (NBIN * 8, NBIN * 8), 1)
        BD = jnp.where((gi // 8 == gj // 8) & (gj < gi), 1.0, 0.0)
        rowtot = _dot(OS, jnp.ones((128, 1), jnp.float32))
        RP = _dot(BD, rowtot)

        li = lax.broadcasted_iota(jnp.int32, (128, 128), 0)
        lj = lax.broadcasted_iota(jnp.int32, (128, 128), 1)
        U = jnp.where(li < lj, 1.0, 0.0)
        LP = _dot(OS, U)

        pos = jnp.zeros((8, 128), jnp.float32)
        cnt = jnp.zeros((1, 128), jnp.float32)
        for b in range(NBIN):
            Ob = OS[8 * b:8 * b + 8]
            contrib = Ob * (RP[8 * b:8 * b + 8] + LP[8 * b:8 * b + 8])
            pos = pos + contrib
            tot = jnp.sum(Ob)
            oh = jnp.where(
                lax.broadcasted_iota(jnp.int32, (1, 128), 1) == b, 1.0, 0.0)
            cnt = cnt + tot * oh
        pl_o[...] = pos.astype(jnp.int32)
        c_o[...] = cnt[:, :32].reshape(1, 1, 32)

    return pl.pallas_call(
        body,
        grid=grid,
        in_specs=[pl.BlockSpec((8, 128), lambda i: (i, 0))],
        out_specs=[
            pl.BlockSpec((8, 128), lambda i: (i, 0)),
            pl.BlockSpec((1, 1, 32), lambda i: (i, 0, 0)),
        ],
        out_shape=[
            jax.ShapeDtypeStruct((NR, 128), jnp.int32),
            jax.ShapeDtypeStruct((NR // 8, 1, 32), jnp.float32),
        ],
    )(idx2)


def _r2b(C):
    NB = C.shape[0]

    def body(c_r, bm_o, st_o):
        Cv = c_r[...].reshape(NB, 32)
        tot = jnp.sum(Cv, axis=0, keepdims=True)
        bi = lax.broadcasted_iota(jnp.int32, (32, 32), 0)
        bj = lax.broadcasted_iota(jnp.int32, (32, 32), 1)
        U32 = jnp.where(bi < bj, 1.0, 0.0)
        start = _dot(tot, U32)
        ri = lax.broadcasted_iota(jnp.int32, (NB, NB), 0)
        rj = lax.broadcasted_iota(jnp.int32, (NB, NB), 1)
        UB = jnp.where(rj < ri, 1.0, 0.0)
        blkpfx = _dot(UB, Cv)
        bm_o[...] = (blkpfx + start).reshape(NB, 1, 32)
        st_o[...] = start.astype(jnp.int32)

    return pl.pallas_call(
        body,
        in_specs=[_full(C)],
        out_specs=[
            pl.BlockSpec((NB, 1, 32), lambda: (0, 0, 0)),
            pl.BlockSpec((1, 32), lambda: (0, 0)),
        ],
        out_shape=[
            jax.ShapeDtypeStruct((NB, 1, 32), jnp.float32),
            jax.ShapeDtypeStruct((1, 32), jnp.int32),
        ],
    )(C)


def _r2c(idx2, pos_local, Bm):
    NR = idx2.shape[0]
    grid = (NR // 8,)

    def body(ix_r, pl_r, bm_r, pos_o):
        d = ix_r[...] >> 13
        pos = pl_r[...].astype(jnp.float32)
        for b in range(NBIN):
            Ob = jnp.where(d == b, 1.0, 0.0)
            pos = pos + Ob * bm_r[0, 0, b]
        pos_o[...] = pos.astype(jnp.int32)

    return pl.pallas_call(
        body,
        grid=grid,
        in_specs=[
            pl.BlockSpec((8, 128), lambda i: (i, 0)),
            pl.BlockSpec((8, 128), lambda i: (i, 0)),
            pl.BlockSpec((1, 1, 32), lambda i: (i, 0, 0)),
        ],
        out_specs=pl.BlockSpec((8, 128), lambda i: (i, 0)),
        out_shape=jax.ShapeDtypeStruct((NR, 128), jnp.int32),
    )(idx2, pos_local, Bm)


def _k4a(idx2, pos, zeros_i):
    NR = idx2.shape[0]
    T = NR * 128
    NW = NC * NS
    RPW = NR // NW
    REM = NR - RPW * NW
    WPT = T // NS
    ZB = 4000
    mesh = plsc.VectorSubcoreMesh(core_axis_name="c", subcore_axis_name="s")

    @functools.partial(
        pl.kernel,
        mesh=mesh,
        out_type=[
            jax.ShapeDtypeStruct((NC * T + 4096,), jnp.int32),
            jax.ShapeDtypeStruct((NC * T + 4096,), jnp.int32),
        ],
        scratch_types=[
            pltpu.VMEM((1, 128), jnp.int32),
            pltpu.VMEM((1, 128), jnp.int32),
            pltpu.VMEM((1, 128), jnp.int32),
            pltpu.VMEM((ZB,), jnp.int32),
            pltpu.VMEM((ZB,), jnp.int32),
            pltpu.VMEM_SHARED((T,), jnp.int32),
            pltpu.VMEM_SHARED((T,), jnp.int32),
        ],
    )
    def k(ix_hbm, pos_hbm, z_hbm, tid_hbm, dst_hbm,
          posb, valb, tidb, zb, sbuf, tidS, dstS):
        c = lax.axis_index("c")
        s = lax.axis_index("s")
        w = s * NC + c

        pltpu.sync_copy(z_hbm, zb)

        def zero(j, _):
            pltpu.sync_copy(zb, tidS.at[pl.ds(s * WPT + j * ZB, ZB)])
            pltpu.sync_copy(zb, dstS.at[pl.ds(s * WPT + j * ZB, ZB)])
            return 0
        lax.fori_loop(0, WPT // ZB, zero, 0)
        plsc.subcore_barrier()

        def do_row(row, _):
            pltpu.sync_copy(pos_hbm.at[row], posb.at[0])
            pltpu.sync_copy(ix_hbm.at[row], valb.at[0])
            for j in range(8):
                tidb[0, pl.ds(16 * j, 16)] = (
                    row * 128 + 16 * j + lax.iota(jnp.int32, 16))
            pltpu.sync_copy(valb.at[0], dstS.at[posb.at[0]], add=True)
            pltpu.sync_copy(tidb.at[0], tidS.at[posb.at[0]], add=True)
            return 0

        def loop(i, _):
            do_row(w * RPW + i, 0)
            return 0
        lax.fori_loop(0, RPW, loop, 0)

        @pl.when(w < REM)
        def _():
            do_row(NW * RPW + w, 0)

        plsc.subcore_barrier()

        def dump(j, _):
            off = s * WPT + j * ZB
            pltpu.sync_copy(tidS.at[pl.ds(off, ZB)], sbuf)
            pltpu.sync_copy(sbuf, tid_hbm.at[pl.ds(c * T + off, ZB)])
            pltpu.sync_copy(dstS.at[pl.ds(off, ZB)], sbuf)
            pltpu.sync_copy(sbuf, dst_hbm.at[pl.ds(c * T + off, ZB)])
            return 0
        lax.fori_loop(0, WPT // ZB, dump, 0)

    return k(idx2, pos, zeros_i)


def _k4b(mu, tidP, dstP, starti, zeros4):
    T = mu.shape[0]
    H = mu.shape[1]
    ACCR = 8448
    SB = 128
    CH = 16
    CR = CH * SB
    mesh = plsc.VectorSubcoreMesh(core_axis_name="c", subcore_axis_name="s")

    @functools.partial(
        pl.kernel,
        mesh=mesh,
        out_type=jax.ShapeDtypeStruct((NBIN * PRB, H), jnp.float32),
        scratch_types=[
            pltpu.VMEM((1, 32), jnp.int32),
            pltpu.VMEM((CR,), jnp.int32),
            pltpu.VMEM((CR,), jnp.int32),
            pltpu.VMEM((1, CR), jnp.int32),
            pltpu.VMEM((CR,), jnp.int32),
            pltpu.VMEM((CR,), jnp.int32),
            pltpu.VMEM((1, SB), jnp.int32),
            pltpu.VMEM((SB, H), jnp.float32),
            pltpu.VMEM((16, H), jnp.float32),
            pltpu.VMEM_SHARED((ACCR, H), jnp.float32),
            pltpu.SemaphoreType.DMA,
        ],
    )
    def k(mu_hbm, tid_hbm, dst_hbm, st_hbm, z_hbm, seg_hbm,
          startv, tA, tB, tC, dA, dB, drow, gbuf, zbuf, acc, sem):
        c = lax.axis_index("c")
        s = lax.axis_index("s")
        pltpu.sync_copy(z_hbm, zbuf)
        pltpu.sync_copy(st_hbm, startv)
        v0 = startv[0, pl.ds(0, 16)]
        v1 = startv[0, pl.ds(16, 16)]

        def get_start(kk):
            return v0[kk] if kk < 16 else v1[kk - 16]

        for p in range(NBPC):
            st_a, en_a = get_start(p), get_start(p + 1)
            st_b, en_b = get_start(10 + p), get_start(11 + p)
            st = st_a * (1 - c) + st_b * c
            en = en_a * (1 - c) + en_b * c
            st = jnp.minimum(jnp.maximum(st, 0), T)
            en = jnp.minimum(jnp.maximum(en, st), T)
            lo = (c * NBPC + p) * PRB

            def zero(j, _):
                pltpu.sync_copy(zbuf, acc.at[pl.ds(s * 528 + j * 16, 16)])
                return 0
            lax.fori_loop(0, 33, zero, 0)
            plsc.subcore_barrier()

            b0 = (st >> 7)
            nb = ((en + SB - 1) >> 7) - b0
            span = (nb + NS - 1) // NS
            myb = b0 + s * span
            myn = jnp.minimum(jnp.maximum(nb - s * span, 0), span)
            nch = (myn + CH - 1) // CH

            def chunk(q, _):
                t0c = (myb + q * CH) * SB
                pltpu.sync_copy(tid_hbm.at[pl.ds(t0c, CR)], tA)
                pltpu.sync_copy(tid_hbm.at[pl.ds(T + t0c, CR)], tB)
                pltpu.sync_copy(dst_hbm.at[pl.ds(t0c, CR)], dA)
                pltpu.sync_copy(dst_hbm.at[pl.ds(T + t0c, CR)], dB)
                nin = jnp.minimum(myn - q * CH, CH)

                def batch(j, _):
                    off = j * SB
                    for m in range(SB // 16):
                        tv = (tA[pl.ds(off + 16 * m, 16)]
                              + tB[pl.ds(off + 16 * m, 16)])
                        tC[0, pl.ds(off + 16 * m, 16)] = jnp.minimum(
                            jnp.maximum(tv, 0), T - 1)
                    pltpu.async_copy(
                        mu_hbm.at[tC.at[0, pl.ds(off, SB)]],
                        gbuf, sem).wait()
                    for m in range(SB // 16):
                        v = (dA[pl.ds(off + 16 * m, 16)]
                             + dB[pl.ds(off + 16 * m, 16)])
                        oc = jnp.minimum(jnp.maximum(v - lo, -8), PRB) + 8
                        drow[0, pl.ds(16 * m, 16)] = oc
                    pltpu.sync_copy(gbuf, acc.at[drow.at[0]], add=True)
                    return 0
                lax.fori_loop(0, nin, batch, 0)
                return 0
            lax.fori_loop(0, nch, chunk, 0)
            plsc.subcore_barrier()

            pltpu.sync_copy(acc.at[pl.ds(8 + s * 512, 512)],
                            seg_hbm.at[pl.ds(lo + s * 512, 512)])
            plsc.subcore_barrier()

    return k(mu, tidP, dstP, starti, zeros4)


def _k5(seg, x_ji, x, orbf, bW, bB, W_lin, b_lin, aW, aB):
    E, H = x.shape
    BE = 640
    grid = (E // BE,)

    def body(seg_r, xji_r, x_r, orbf_r, bW_r, bB_r, Wl_r, bl_r,
             aW_r, aB_r, t_o):
        h = xji_r[...] + _silu(seg_r[...])
        for l in range(bW_r.shape[0]):
            u = _silu(_dot(h, bW_r[l, 0]) + bB_r[l, 0])
            h = h + _silu(_dot(u, bW_r[l, 1]) + bB_r[l, 1])
        h = _silu(_dot(h, Wl_r[...]) + bl_r[...]) + x_r[...]
        for l in range(aW_r.shape[0]):
            u = _silu(_dot(h, aW_r[l, 0]) + aB_r[l, 0])
            h = h + _silu(_dot(u, aW_r[l, 1]) + aB_r[l, 1])
        t_o[...] = orbf_r[...] * h

    return pl.pallas_call(
        body,
        grid=grid,
        in_specs=[
            pl.BlockSpec((BE, H), lambda i: (i, 0)),
            pl.BlockSpec((BE, H), lambda i: (i, 0)),
            pl.BlockSpec((BE, H), lambda i: (i, 0)),
            pl.BlockSpec((BE, H), lambda i: (i, 0)),
            _full(bW), _full(bB), _full(W_lin), _full(b_lin),
            _full(aW), _full(aB),
        ],
        out_specs=pl.BlockSpec((BE, H), lambda i: (i, 0)),
        out_shape=jax.ShapeDtypeStruct((E, H), jnp.float32),
    )(seg, x_ji, x, orbf, bW, bB, W_lin, b_lin, aW, aB)


def _k6_node_scatter(t_arr, idx_i, zeros6):
    E, H = t_arr.shape
    EPC = E // NC
    EPT = EPC // NS
    SB = 128
    NFULL = EPT // SB
    TAIL = EPT - NFULL * SB
    ACCR = 10240
    mesh = plsc.VectorSubcoreMesh(core_axis_name="c", subcore_axis_name="s")

    @functools.partial(
        pl.kernel,
        mesh=mesh,
        out_type=jax.ShapeDtypeStruct((NC, N_NODES, H), jnp.float32),
        scratch_types=[
            pltpu.VMEM((NFULL + 1, SB), jnp.int32),
            pltpu.VMEM((SB, H), jnp.float32),
            pltpu.VMEM((TAIL, H), jnp.float32),
            pltpu.VMEM((16, H), jnp.float32),
            pltpu.VMEM_SHARED((ACCR, H), jnp.float32),
            pltpu.SemaphoreType.DMA,
        ],
    )
    def k(t_hbm, idx_hbm, z_hbm, part_hbm, idx_v, mbuf, tbuf, zbuf, acc, sem):
        c = lax.axis_index("c")
        s = lax.axis_index("s")
        ebase = c * EPC + s * EPT

        def load_idx(b, _):
            pltpu.sync_copy(idx_hbm.at[pl.ds(ebase + b * SB, SB)], idx_v.at[b])
            return 0
        lax.fori_loop(0, NFULL, load_idx, 0)
        pltpu.sync_copy(idx_hbm.at[pl.ds(ebase + NFULL * SB, TAIL)],
                        idx_v.at[NFULL, pl.ds(0, TAIL)])

        pltpu.sync_copy(z_hbm, zbuf)

        def zero(j, _):
            pltpu.sync_copy(zbuf, acc.at[pl.ds(s * 640 + j * 16, 16)])
            return 0
        lax.fori_loop(0, 40, zero, 0)
        plsc.subcore_barrier()

        def scat(b, _):
            pltpu.sync_copy(t_hbm.at[pl.ds(ebase + b * SB, SB)], mbuf)
            pltpu.sync_copy(mbuf, acc.at[idx_v.at[b]], add=True)
            return 0
        lax.fori_loop(0, NFULL, scat, 0)
        pltpu.sync_copy(t_hbm.at[pl.ds(ebase + NFULL * SB, TAIL)], tbuf)
        pltpu.sync_copy(tbuf, acc.at[idx_v.at[NFULL, pl.ds(0, TAIL)]],
                        add=True)
        plsc.subcore_barrier()

        pltpu.sync_copy(acc.at[pl.ds(s * 624, 624)],
                        part_hbm.at[c, pl.ds(s * 624, 624)])

        @pl.when(s == NS - 1)
        def _():
            pltpu.sync_copy(acc.at[pl.ds(9984, 16)],
                            part_hbm.at[c, pl.ds(9984, 16)])

    return k(t_arr, idx_i, zeros6)


def _k7(part, W_oup, b_oup, oW, oB, W_out):
    H = part.shape[2]
    OC = W_out.shape[1]
    BN = 2000
    grid = (N_NODES // BN,)

    def body(p_r, Wo_r, bo_r, oW_r, oB_r, Wout_r, out_o):
        tt = p_r[0] + p_r[1]
        y = _dot(tt, Wo_r[...]) + bo_r[...]
        for l in range(oW_r.shape[0]):
            y = _silu(_dot(y, oW_r[l]) + oB_r[l])
        out_o[...] = _dot(y, Wout_r[...])

    return pl.pallas_call(
        body,
        grid=grid,
        in_specs=[
            pl.BlockSpec((NC, BN, H), lambda i: (0, i, 0)),
            _full(W_oup), _full(b_oup), _full(oW), _full(oB), _full(W_out),
        ],
        out_specs=pl.BlockSpec((BN, OC), lambda i: (i, 0)),
        out_shape=jax.ShapeDtypeStruct((N_NODES, OC), jnp.float32),
    )(part, W_oup, b_oup, oW, oB, W_out)


def kernel(x, rbf, sbf, idx_kj, idx_ji, idx_i,
           W_ji, b_ji, W_kj, b_kj, W_rbf1, W_rbf2, W_sbf1, W_sbf2,
           W_down, W_up, bW, bB, W_lin, b_lin, aW, aB,
           W_orbf, W_oup, b_oup, oW, oB, W_out):
    H = x.shape[1]
    T = idx_ji.shape[0]
    zeros16 = jnp.zeros((16, H), jnp.float32)

    x_ji, xkj_mid, orbf = _k1(x, rbf, W_ji, b_ji, W_kj, b_kj,
                              W_rbf1, W_rbf2, W_orbf)
    g = _k2_gather(xkj_mid, idx_kj)
    mu = _k3(sbf, g, W_sbf1, W_sbf2, W_down, W_up)

    idx2 = idx_ji.reshape(T // 128, 128)
    pos_local, C = _r2a(idx2)
    Bm, starti = _r2b(C)
    pos = _r2c(idx2, pos_local, Bm)
    zeros_i = jnp.zeros((4000,), jnp.int32)
    tidP, dstP = _k4a(idx2, pos, zeros_i)
    seg_pad = _k4b(mu, tidP, dstP, starti, zeros16)

    t_arr = _k5(seg_pad, x_ji, x, orbf, bW, bB, W_lin, b_lin, aW, aB)
    part = _k6_node_scatter(t_arr, idx_i, zeros16)
    return _k7(part, W_oup, b_oup, oW, oB, W_out)


# speedup vs baseline: 4.4332x; 1.0180x over previous
import functools

import jax
import jax.numpy as jnp
from jax import lax
from jax.experimental import pallas as pl
from jax.experimental.pallas import tpu as pltpu
from jax.experimental.pallas import tpu_sc as plsc

N_NODES = 10000
NC = 2
NS = 16
PRB = 8192
NBIN = 20
NBPC = 10


def _silu(v):
    return v * (1.0 / (1.0 + jnp.exp(-v)))


def _dot(a, b):
    return jnp.dot(a, b, preferred_element_type=jnp.float32)


def _full(a):
    return pl.BlockSpec(a.shape, lambda *args: (0,) * a.ndim)


def _k1(x, rbf, W_ji, b_ji, W_kj, b_kj, W_rbf1, W_rbf2, W_orbf):
    E, H = x.shape
    R = rbf.shape[1]
    BE = 640
    grid = (E // BE,)

    def body(x_r, rbf_r, Wji_r, bji_r, Wkj_r, bkj_r, Wr1_r, Wr2_r,
             Wo_r, xji_o, xkj_o, orbf_o):
        xb = x_r[...]
        rb = rbf_r[...]
        xji_o[...] = _silu(_dot(xb, Wji_r[...]) + bji_r[...])
        rbf_e = _dot(_dot(rb, Wr1_r[...]), Wr2_r[...])
        xkj_o[...] = _silu(_dot(xb, Wkj_r[...]) + bkj_r[...]) * rbf_e
        orbf_o[...] = _dot(rb, Wo_r[...])

    return pl.pallas_call(
        body,
        grid=grid,
        in_specs=[
            pl.BlockSpec((BE, H), lambda i: (i, 0)),
            pl.BlockSpec((BE, R), lambda i: (i, 0)),
            _full(W_ji), _full(b_ji), _full(W_kj), _full(b_kj),
            _full(W_rbf1), _full(W_rbf2), _full(W_orbf),
        ],
        out_specs=[
            pl.BlockSpec((BE, H), lambda i: (i, 0)),
            pl.BlockSpec((BE, H), lambda i: (i, 0)),
            pl.BlockSpec((BE, H), lambda i: (i, 0)),
        ],
        out_shape=[
            jax.ShapeDtypeStruct((E, H), jnp.float32),
            jax.ShapeDtypeStruct((E, H), jnp.float32),
            jax.ShapeDtypeStruct((E, H), jnp.float32),
        ],
    )(x, rbf, W_ji, b_ji, W_kj, b_kj, W_rbf1, W_rbf2, W_orbf)


def _k2_gather(table, idx_kj):
    E, H = table.shape
    T = idx_kj.shape[0]
    NW = NC * NS
    TPW = T // NW
    SB = 128
    NFULL = TPW // SB
    TAIL = TPW - NFULL * SB
    mesh = plsc.VectorSubcoreMesh(core_axis_name="c", subcore_axis_name="s")

    @functools.partial(
        pl.kernel,
        mesh=mesh,
        out_type=jax.ShapeDtypeStruct((T, H), jnp.float32),
        scratch_types=[
            pltpu.VMEM((NFULL + 1, SB), jnp.int32),
            pltpu.VMEM((SB, H), jnp.float32),
            pltpu.VMEM((SB, H), jnp.float32),
            pltpu.VMEM((TAIL, H), jnp.float32),
            pltpu.SemaphoreType.DMA,
            pltpu.SemaphoreType.DMA,
        ],
    )
    def k(tab_hbm, idx_hbm, g_hbm, idx_v, rows_v, rows2_v, tail_v, sem,
          sem2):
        c = lax.axis_index("c")
        s = lax.axis_index("s")
        base = (s * NC + c) * TPW

        def load_idx(b, _):
            pltpu.sync_copy(idx_hbm.at[pl.ds(base + b * SB, SB)], idx_v.at[b])
            return 0
        lax.fori_loop(0, NFULL, load_idx, 0)
        pltpu.sync_copy(idx_hbm.at[pl.ds(base + NFULL * SB, TAIL)],
                        idx_v.at[NFULL, pl.ds(0, TAIL)])

        def gath2(q, _):
            b0, b1 = 2 * q, 2 * q + 1
            d0 = pltpu.async_copy(tab_hbm.at[idx_v.at[b0]], rows_v, sem)
            d1 = pltpu.async_copy(tab_hbm.at[idx_v.at[b1]], rows2_v, sem2)
            d0.wait()
            pltpu.sync_copy(rows_v, g_hbm.at[pl.ds(base + b0 * SB, SB)])
            d1.wait()
            pltpu.sync_copy(rows2_v, g_hbm.at[pl.ds(base + b1 * SB, SB)])
            return 0
        lax.fori_loop(0, NFULL // 2, gath2, 0)
        pltpu.async_copy(tab_hbm.at[idx_v.at[NFULL, pl.ds(0, TAIL)]],
                         tail_v, sem).wait()
        pltpu.sync_copy(tail_v, g_hbm.at[pl.ds(base + NFULL * SB, TAIL)])

    return k(table, idx_kj)


def _k3(sbf, g, W_sbf1, W_sbf2, W_down, W_up):
    T, SR = sbf.shape
    H = g.shape[1]
    BT = 1024
    grid = (T // BT,)

    def body(sbf_r, g_r, W1_r, W2_r, Wd_r, Wu_r, mu_o):
        z = _dot(_dot(sbf_r[...], W1_r[...]), W2_r[...])
        xkd = _silu(_dot(g_r[...], Wd_r[...]))
        mu_o[...] = _dot(xkd * z, Wu_r[...])

    return pl.pallas_call(
        body,
        grid=grid,
        in_specs=[
            pl.BlockSpec((BT, SR), lambda i: (i, 0)),
            pl.BlockSpec((BT, H), lambda i: (i, 0)),
            _full(W_sbf1), _full(W_sbf2), _full(W_down), _full(W_up),
        ],
        out_specs=pl.BlockSpec((BT, H), lambda i: (i, 0)),
        out_shape=jax.ShapeDtypeStruct((T, H), jnp.float32),
    )(sbf, g, W_sbf1, W_sbf2, W_down, W_up)


def _r2a(idx2):
    NR = idx2.shape[0]
    grid = (NR // 8,)

    def body(ix_r, pl_o, c_o):
        d = ix_r[...] >> 13
        rows = []
        for b in range(NBIN):
            rows.append(jnp.where(d == b, 1.0, 0.0))
        OS = jnp.concatenate(rows, axis=0)

        gi = lax.broadcasted_iota(jnp.int32, (NBIN * 8, NBIN * 8), 0)
        gj = lax.broadcasted_iota(jnp.int32, (NBIN * 8, NBIN * 8), 1)
        BD = jnp.where((gi // 8 == gj // 8) & (gj < gi), 1.0, 0.0)
        rowtot = _dot(OS, jnp.ones((128, 1), jnp.float32))
        RP = _dot(BD, rowtot)

        li = lax.broadcasted_iota(jnp.int32, (128, 128), 0)
        lj = lax.broadcasted_iota(jnp.int32, (128, 128), 1)
        U = jnp.where(li < lj, 1.0, 0.0)
        LP = _dot(OS, U)

        pos = jnp.zeros((8, 128), jnp.float32)
        cnt = jnp.zeros((1, 128), jnp.float32)
        for b in range(NBIN):
            Ob = OS[8 * b:8 * b + 8]
            contrib = Ob * (RP[8 * b:8 * b + 8] + LP[8 * b:8 * b + 8])
            pos = pos + contrib
            tot = jnp.sum(Ob)
            oh = jnp.where(
                lax.broadcasted_iota(jnp.int32, (1, 128), 1) == b, 1.0, 0.0)
            cnt = cnt + tot * oh
        pl_o[...] = pos.astype(jnp.int32)
        c_o[...] = cnt[:, :32].reshape(1, 1, 32)

    return pl.pallas_call(
        body,
        grid=grid,
        in_specs=[pl.BlockSpec((8, 128), lambda i: (i, 0))],
        out_specs=[
            pl.BlockSpec((8, 128), lambda i: (i, 0)),
            pl.BlockSpec((1, 1, 32), lambda i: (i, 0, 0)),
        ],
        out_shape=[
            jax.ShapeDtypeStruct((NR, 128), jnp.int32),
            jax.ShapeDtypeStruct((NR // 8, 1, 32), jnp.float32),
        ],
    )(idx2)


def _r2b(C):
    NB = C.shape[0]

    def body(c_r, bm_o, st_o):
        Cv = c_r[...].reshape(NB, 32)
        tot = jnp.sum(Cv, axis=0, keepdims=True)
        bi = lax.broadcasted_iota(jnp.int32, (32, 32), 0)
        bj = lax.broadcasted_iota(jnp.int32, (32, 32), 1)
        U32 = jnp.where(bi < bj, 1.0, 0.0)
        start = _dot(tot, U32)
        ri = lax.broadcasted_iota(jnp.int32, (NB, NB), 0)
        rj = lax.broadcasted_iota(jnp.int32, (NB, NB), 1)
        UB = jnp.where(rj < ri, 1.0, 0.0)
        blkpfx = _dot(UB, Cv)
        bm_o[...] = (blkpfx + start).reshape(NB, 1, 32)
        st_o[...] = start.astype(jnp.int32)

    return pl.pallas_call(
        body,
        in_specs=[_full(C)],
        out_specs=[
            pl.BlockSpec((NB, 1, 32), lambda: (0, 0, 0)),
            pl.BlockSpec((1, 32), lambda: (0, 0)),
        ],
        out_shape=[
            jax.ShapeDtypeStruct((NB, 1, 32), jnp.float32),
            jax.ShapeDtypeStruct((1, 32), jnp.int32),
        ],
    )(C)


def _r2c(idx2, pos_local, Bm):
    NR = idx2.shape[0]
    grid = (NR // 8,)

    def body(ix_r, pl_r, bm_r, pos_o):
        d = ix_r[...] >> 13
        pos = pl_r[...].astype(jnp.float32)
        for b in range(NBIN):
            Ob = jnp.where(d == b, 1.0, 0.0)
            pos = pos + Ob * bm_r[0, 0, b]
        pos_o[...] = pos.astype(jnp.int32)

    return pl.pallas_call(
        body,
        grid=grid,
        in_specs=[
            pl.BlockSpec((8, 128), lambda i: (i, 0)),
            pl.BlockSpec((8, 128), lambda i: (i, 0)),
            pl.BlockSpec((1, 1, 32), lambda i: (i, 0, 0)),
        ],
        out_specs=pl.BlockSpec((8, 128), lambda i: (i, 0)),
        out_shape=jax.ShapeDtypeStruct((NR, 128), jnp.int32),
    )(idx2, pos_local, Bm)


def _k4a(idx2, pos, zeros_i):
    NR = idx2.shape[0]
    T = NR * 128
    NW = NC * NS
    RPW = NR // NW
    REM = NR - RPW * NW
    WPT = T // NS
    ZB = 4000
    mesh = plsc.VectorSubcoreMesh(core_axis_name="c", subcore_axis_name="s")

    @functools.partial(
        pl.kernel,
        mesh=mesh,
        out_type=[
            jax.ShapeDtypeStruct((NC * T + 4096,), jnp.int32),
            jax.ShapeDtypeStruct((NC * T + 4096,), jnp.int32),
        ],
        scratch_types=[
            pltpu.VMEM((1, 128), jnp.int32),
            pltpu.VMEM((1, 128), jnp.int32),
            pltpu.VMEM((1, 128), jnp.int32),
            pltpu.VMEM((ZB,), jnp.int32),
            pltpu.VMEM((ZB,), jnp.int32),
            pltpu.VMEM_SHARED((T,), jnp.int32),
            pltpu.VMEM_SHARED((T,), jnp.int32),
        ],
    )
    def k(ix_hbm, pos_hbm, z_hbm, tid_hbm, dst_hbm,
          posb, valb, tidb, zb, sbuf, tidS, dstS):
        c = lax.axis_index("c")
        s = lax.axis_index("s")
        w = s * NC + c

        pltpu.sync_copy(z_hbm, zb)

        def zero(j, _):
            pltpu.sync_copy(zb, tidS.at[pl.ds(s * WPT + j * ZB, ZB)])
            pltpu.sync_copy(zb, dstS.at[pl.ds(s * WPT + j * ZB, ZB)])
            return 0
        lax.fori_loop(0, WPT // ZB, zero, 0)
        plsc.subcore_barrier()

        def do_row(row, _):
            pltpu.sync_copy(pos_hbm.at[row], posb.at[0])
            pltpu.sync_copy(ix_hbm.at[row], valb.at[0])
            for j in range(8):
                tidb[0, pl.ds(16 * j, 16)] = (
                    row * 128 + 16 * j + lax.iota(jnp.int32, 16))
            pltpu.sync_copy(valb.at[0], dstS.at[posb.at[0]], add=True)
            pltpu.sync_copy(tidb.at[0], tidS.at[posb.at[0]], add=True)
            return 0

        def loop(i, _):
            do_row(w * RPW + i, 0)
            return 0
        lax.fori_loop(0, RPW, loop, 0)

        @pl.when(w < REM)
        def _():
            do_row(NW * RPW + w, 0)

        plsc.subcore_barrier()

        def dump(j, _):
            off = s * WPT + j * ZB
            pltpu.sync_copy(tidS.at[pl.ds(off, ZB)], sbuf)
            pltpu.sync_copy(sbuf, tid_hbm.at[pl.ds(c * T + off, ZB)])
            pltpu.sync_copy(dstS.at[pl.ds(off, ZB)], sbuf)
            pltpu.sync_copy(sbuf, dst_hbm.at[pl.ds(c * T + off, ZB)])
            return 0
        lax.fori_loop(0, WPT // ZB, dump, 0)

    return k(idx2, pos, zeros_i)


def _k4b(mu, tidP, dstP, starti, zeros4):
    T = mu.shape[0]
    H = mu.shape[1]
    ACCR = 8448
    SB = 128
    CH = 16
    CR = CH * SB
    mesh = plsc.VectorSubcoreMesh(core_axis_name="c", subcore_axis_name="s")

    @functools.partial(
        pl.kernel,
        mesh=mesh,
        out_type=jax.ShapeDtypeStruct((NBIN * PRB, H), jnp.float32),
        scratch_types=[
            pltpu.VMEM((1, 32), jnp.int32),
            pltpu.VMEM((CR,), jnp.int32),
            pltpu.VMEM((CR,), jnp.int32),
            pltpu.VMEM((1, CR), jnp.int32),
            pltpu.VMEM((CR,), jnp.int32),
            pltpu.VMEM((CR,), jnp.int32),
            pltpu.VMEM((1, SB), jnp.int32),
            pltpu.VMEM((SB, H), jnp.float32),
            pltpu.VMEM((SB, H), jnp.float32),
            pltpu.VMEM((16, H), jnp.float32),
            pltpu.VMEM_SHARED((ACCR, H), jnp.float32),
            pltpu.SemaphoreType.DMA,
            pltpu.SemaphoreType.DMA,
        ],
    )
    def k(mu_hbm, tid_hbm, dst_hbm, st_hbm, z_hbm, seg_hbm,
          startv, tA, tB, tC, dA, dB, drow, gbuf, gbuf2, zbuf, acc, sem,
          sem2):
        c = lax.axis_index("c")
        s = lax.axis_index("s")
        pltpu.sync_copy(z_hbm, zbuf)
        pltpu.sync_copy(st_hbm, startv)
        v0 = startv[0, pl.ds(0, 16)]
        v1 = startv[0, pl.ds(16, 16)]

        def get_start(kk):
            return v0[kk] if kk < 16 else v1[kk - 16]

        for p in range(NBPC):
            st_a, en_a = get_start(p), get_start(p + 1)
            st_b, en_b = get_start(10 + p), get_start(11 + p)
            st = st_a * (1 - c) + st_b * c
            en = en_a * (1 - c) + en_b * c
            st = jnp.minimum(jnp.maximum(st, 0), T)
            en = jnp.minimum(jnp.maximum(en, st), T)
            lo = (c * NBPC + p) * PRB

            def zero(j, _):
                pltpu.sync_copy(zbuf, acc.at[pl.ds(s * 528 + j * 16, 16)])
                return 0
            lax.fori_loop(0, 33, zero, 0)
            plsc.subcore_barrier()

            b0 = (st >> 7)
            nb = ((en + SB - 1) >> 7) - b0
            span = (nb + NS - 1) // NS
            myb = b0 + s * span
            myn = jnp.minimum(jnp.maximum(nb - s * span, 0), span)
            nch = (myn + CH - 1) // CH

            def chunk(q, _):
                t0c = (myb + q * CH) * SB
                pltpu.sync_copy(tid_hbm.at[pl.ds(t0c, CR)], tA)
                pltpu.sync_copy(tid_hbm.at[pl.ds(T + t0c, CR)], tB)
                pltpu.sync_copy(dst_hbm.at[pl.ds(t0c, CR)], dA)
                pltpu.sync_copy(dst_hbm.at[pl.ds(T + t0c, CR)], dB)
                nin = jnp.minimum(myn - q * CH, CH)

                def mk_tC(j):
                    off = j * SB
                    for m in range(SB // 16):
                        tv = (tA[pl.ds(off + 16 * m, 16)]
                              + tB[pl.ds(off + 16 * m, 16)])
                        tC[0, pl.ds(off + 16 * m, 16)] = jnp.minimum(
                            jnp.maximum(tv, 0), T - 1)
                    return off

                def scat(j, gb):
                    off = j * SB
                    for m in range(SB // 16):
                        v = (dA[pl.ds(off + 16 * m, 16)]
                             + dB[pl.ds(off + 16 * m, 16)])
                        oc = jnp.minimum(jnp.maximum(v - lo, -8), PRB) + 8
                        drow[0, pl.ds(16 * m, 16)] = oc
                    pltpu.sync_copy(gb, acc.at[drow.at[0]], add=True)

                def pair(j2, _):
                    j0 = 2 * j2
                    o0 = mk_tC(j0)
                    d0 = pltpu.async_copy(
                        mu_hbm.at[tC.at[0, pl.ds(o0, SB)]], gbuf, sem)
                    o1 = mk_tC(j0 + 1)
                    d1 = pltpu.async_copy(
                        mu_hbm.at[tC.at[0, pl.ds(o1, SB)]], gbuf2, sem2)
                    d0.wait()
                    scat(j0, gbuf)
                    d1.wait()
                    scat(j0 + 1, gbuf2)
                    return 0
                lax.fori_loop(0, nin >> 1, pair, 0)

                @pl.when((nin & 1) == 1)
                def _():
                    jl = nin - 1
                    ol = mk_tC(jl)
                    pltpu.async_copy(
                        mu_hbm.at[tC.at[0, pl.ds(ol, SB)]],
                        gbuf, sem).wait()
                    scat(jl, gbuf)
                return 0
            lax.fori_loop(0, nch, chunk, 0)
            plsc.subcore_barrier()

            pltpu.sync_copy(acc.at[pl.ds(8 + s * 512, 512)],
                            seg_hbm.at[pl.ds(lo + s * 512, 512)])
            plsc.subcore_barrier()

    return k(mu, tidP, dstP, starti, zeros4)


def _k5(seg, x_ji, x, orbf, bW, bB, W_lin, b_lin, aW, aB):
    E, H = x.shape
    BE = 640
    grid = (E // BE,)

    def body(seg_r, xji_r, x_r, orbf_r, bW_r, bB_r, Wl_r, bl_r,
             aW_r, aB_r, t_o):
        h = xji_r[...] + _silu(seg_r[...])
        for l in range(bW_r.shape[0]):
            u = _silu(_dot(h, bW_r[l, 0]) + bB_r[l, 0])
            h = h + _silu(_dot(u, bW_r[l, 1]) + bB_r[l, 1])
        h = _silu(_dot(h, Wl_r[...]) + bl_r[...]) + x_r[...]
        for l in range(aW_r.shape[0]):
            u = _silu(_dot(h, aW_r[l, 0]) + aB_r[l, 0])
            h = h + _silu(_dot(u, aW_r[l, 1]) + aB_r[l, 1])
        t_o[...] = orbf_r[...] * h

    return pl.pallas_call(
        body,
        grid=grid,
        in_specs=[
            pl.BlockSpec((BE, H), lambda i: (i, 0)),
            pl.BlockSpec((BE, H), lambda i: (i, 0)),
            pl.BlockSpec((BE, H), lambda i: (i, 0)),
            pl.BlockSpec((BE, H), lambda i: (i, 0)),
            _full(bW), _full(bB), _full(W_lin), _full(b_lin),
            _full(aW), _full(aB),
        ],
        out_specs=pl.BlockSpec((BE, H), lambda i: (i, 0)),
        out_shape=jax.ShapeDtypeStruct((E, H), jnp.float32),
    )(seg, x_ji, x, orbf, bW, bB, W_lin, b_lin, aW, aB)


def _k6_node_scatter(t_arr, idx_i, zeros6):
    E, H = t_arr.shape
    EPC = E // NC
    EPT = EPC // NS
    SB = 128
    NFULL = EPT // SB
    TAIL = EPT - NFULL * SB
    ACCR = 10240
    mesh = plsc.VectorSubcoreMesh(core_axis_name="c", subcore_axis_name="s")

    @functools.partial(
        pl.kernel,
        mesh=mesh,
        out_type=jax.ShapeDtypeStruct((NC, N_NODES, H), jnp.float32),
        scratch_types=[
            pltpu.VMEM((NFULL + 1, SB), jnp.int32),
            pltpu.VMEM((SB, H), jnp.float32),
            pltpu.VMEM((TAIL, H), jnp.float32),
            pltpu.VMEM((16, H), jnp.float32),
            pltpu.VMEM_SHARED((ACCR, H), jnp.float32),
            pltpu.SemaphoreType.DMA,
        ],
    )
    def k(t_hbm, idx_hbm, z_hbm, part_hbm, idx_v, mbuf, tbuf, zbuf, acc, sem):
        c = lax.axis_index("c")
        s = lax.axis_index("s")
        ebase = c * EPC + s * EPT

        def load_idx(b, _):
            pltpu.sync_copy(idx_hbm.at[pl.ds(ebase + b * SB, SB)], idx_v.at[b])
            return 0
        lax.fori_loop(0, NFULL, load_idx, 0)
        pltpu.sync_copy(idx_hbm.at[pl.ds(ebase + NFULL * SB, TAIL)],
                        idx_v.at[NFULL, pl.ds(0, TAIL)])

        pltpu.sync_copy(z_hbm, zbuf)

        def zero(j, _):
            pltpu.sync_copy(zbuf, acc.at[pl.ds(s * 640 + j * 16, 16)])
            return 0
        lax.fori_loop(0, 40, zero, 0)
        plsc.subcore_barrier()

        def scat(b, _):
            pltpu.sync_copy(t_hbm.at[pl.ds(ebase + b * SB, SB)], mbuf)
            pltpu.sync_copy(mbuf, acc.at[idx_v.at[b]], add=True)
            return 0
        lax.fori_loop(0, NFULL, scat, 0)
        pltpu.sync_copy(t_hbm.at[pl.ds(ebase + NFULL * SB, TAIL)], tbuf)
        pltpu.sync_copy(tbuf, acc.at[idx_v.at[NFULL, pl.ds(0, TAIL)]],
                        add=True)
        plsc.subcore_barrier()

        pltpu.sync_copy(acc.at[pl.ds(s * 624, 624)],
                        part_hbm.at[c, pl.ds(s * 624, 624)])

        @pl.when(s == NS - 1)
        def _():
            pltpu.sync_copy(acc.at[pl.ds(9984, 16)],
                            part_hbm.at[c, pl.ds(9984, 16)])

    return k(t_arr, idx_i, zeros6)


def _k7(part, W_oup, b_oup, oW, oB, W_out):
    H = part.shape[2]
    OC = W_out.shape[1]
    BN = 2000
    grid = (N_NODES // BN,)

    def body(p_r, Wo_r, bo_r, oW_r, oB_r, Wout_r, out_o):
        tt = p_r[0] + p_r[1]
        y = _dot(tt, Wo_r[...]) + bo_r[...]
        for l in range(oW_r.shape[0]):
            y = _silu(_dot(y, oW_r[l]) + oB_r[l])
        out_o[...] = _dot(y, Wout_r[...])

    return pl.pallas_call(
        body,
        grid=grid,
        in_specs=[
            pl.BlockSpec((NC, BN, H), lambda i: (0, i, 0)),
            _full(W_oup), _full(b_oup), _full(oW), _full(oB), _full(W_out),
        ],
        out_specs=pl.BlockSpec((BN, OC), lambda i: (i, 0)),
        out_shape=jax.ShapeDtypeStruct((N_NODES, OC), jnp.float32),
    )(part, W_oup, b_oup, oW, oB, W_out)


def kernel(x, rbf, sbf, idx_kj, idx_ji, idx_i,
           W_ji, b_ji, W_kj, b_kj, W_rbf1, W_rbf2, W_sbf1, W_sbf2,
           W_down, W_up, bW, bB, W_lin, b_lin, aW, aB,
           W_orbf, W_oup, b_oup, oW, oB, W_out):
    H = x.shape[1]
    T = idx_ji.shape[0]
    zeros16 = jnp.zeros((16, H), jnp.float32)

    x_ji, xkj_mid, orbf = _k1(x, rbf, W_ji, b_ji, W_kj, b_kj,
                              W_rbf1, W_rbf2, W_orbf)
    g = _k2_gather(xkj_mid, idx_kj)
    mu = _k3(sbf, g, W_sbf1, W_sbf2, W_down, W_up)

    idx2 = idx_ji.reshape(T // 128, 128)
    pos_local, C = _r2a(idx2)
    Bm, starti = _r2b(C)
    pos = _r2c(idx2, pos_local, Bm)
    zeros_i = jnp.zeros((4000,), jnp.int32)
    tidP, dstP = _k4a(idx2, pos, zeros_i)
    seg_pad = _k4b(mu, tidP, dstP, starti, zeros16)

    t_arr = _k5(seg_pad, x_ji, x, orbf, bW, bB, W_lin, b_lin, aW, aB)
    part = _k6_node_scatter(t_arr, idx_i, zeros16)
    return _k7(part, W_oup, b_oup, oW, oB, W_out)


# speedup vs baseline: 5.2234x; 1.1782x over previous
import functools

import jax
import jax.numpy as jnp
from jax import lax
from jax.experimental import pallas as pl
from jax.experimental.pallas import tpu as pltpu
from jax.experimental.pallas import tpu_sc as plsc

N_NODES = 10000
NC = 2
NS = 16
PRB = 8192
NBIN = 20
NBPC = 10


def _silu(v):
    return v * (1.0 / (1.0 + jnp.exp(-v)))


def _dot(a, b):
    return jnp.dot(a, b, preferred_element_type=jnp.float32)


def _full(a):
    return pl.BlockSpec(a.shape, lambda *args: (0,) * a.ndim)


def _k1(x, rbf, W_ji, b_ji, W_kj, b_kj, W_rbf1, W_rbf2, W_orbf):
    E, H = x.shape
    R = rbf.shape[1]
    BE = 1600
    grid = (E // BE,)

    def body(x_r, rbf_r, Wji_r, bji_r, Wkj_r, bkj_r, Wr1_r, Wr2_r,
             Wo_r, xji_o, xkj_o, orbf_o):
        xb = x_r[...]
        rb = rbf_r[...]
        xji_o[...] = _silu(_dot(xb, Wji_r[...]) + bji_r[...])
        rbf_e = _dot(_dot(rb, Wr1_r[...]), Wr2_r[...])
        xkj_o[...] = _silu(_dot(xb, Wkj_r[...]) + bkj_r[...]) * rbf_e
        orbf_o[...] = _dot(rb, Wo_r[...])

    return pl.pallas_call(
        body,
        grid=grid,
        in_specs=[
            pl.BlockSpec((BE, H), lambda i: (i, 0)),
            pl.BlockSpec((BE, R), lambda i: (i, 0)),
            _full(W_ji), _full(b_ji), _full(W_kj), _full(b_kj),
            _full(W_rbf1), _full(W_rbf2), _full(W_orbf),
        ],
        out_specs=[
            pl.BlockSpec((BE, H), lambda i: (i, 0)),
            pl.BlockSpec((BE, H), lambda i: (i, 0)),
            pl.BlockSpec((BE, H), lambda i: (i, 0)),
        ],
        out_shape=[
            jax.ShapeDtypeStruct((E, H), jnp.float32),
            jax.ShapeDtypeStruct((E, H), jnp.float32),
            jax.ShapeDtypeStruct((E, H), jnp.float32),
        ],
    )(x, rbf, W_ji, b_ji, W_kj, b_kj, W_rbf1, W_rbf2, W_orbf)


def _k2_gather(table, idx_kj):
    E, H = table.shape
    T = idx_kj.shape[0]
    NW = NC * NS
    TPW = T // NW
    SB = 128
    NFULL = TPW // SB
    TAIL = TPW - NFULL * SB
    mesh = plsc.VectorSubcoreMesh(core_axis_name="c", subcore_axis_name="s")

    @functools.partial(
        pl.kernel,
        mesh=mesh,
        out_type=jax.ShapeDtypeStruct((T, H), jnp.float32),
        scratch_types=[
            pltpu.VMEM((NFULL + 1, SB), jnp.int32),
            pltpu.VMEM((SB, H), jnp.float32),
            pltpu.VMEM((SB, H), jnp.float32),
            pltpu.VMEM((TAIL, H), jnp.float32),
            pltpu.SemaphoreType.DMA,
            pltpu.SemaphoreType.DMA,
        ],
    )
    def k(tab_hbm, idx_hbm, g_hbm, idx_v, rows_v, rows2_v, tail_v, sem,
          sem2):
        c = lax.axis_index("c")
        s = lax.axis_index("s")
        base = (s * NC + c) * TPW

        def load_idx(b, _):
            pltpu.sync_copy(idx_hbm.at[pl.ds(base + b * SB, SB)], idx_v.at[b])
            return 0
        lax.fori_loop(0, NFULL, load_idx, 0)
        pltpu.sync_copy(idx_hbm.at[pl.ds(base + NFULL * SB, TAIL)],
                        idx_v.at[NFULL, pl.ds(0, TAIL)])

        def gath2(q, _):
            b0, b1 = 2 * q, 2 * q + 1
            d0 = pltpu.async_copy(tab_hbm.at[idx_v.at[b0]], rows_v, sem)
            d1 = pltpu.async_copy(tab_hbm.at[idx_v.at[b1]], rows2_v, sem2)
            d0.wait()
            pltpu.sync_copy(rows_v, g_hbm.at[pl.ds(base + b0 * SB, SB)])
            d1.wait()
            pltpu.sync_copy(rows2_v, g_hbm.at[pl.ds(base + b1 * SB, SB)])
            return 0
        lax.fori_loop(0, NFULL // 2, gath2, 0)
        pltpu.async_copy(tab_hbm.at[idx_v.at[NFULL, pl.ds(0, TAIL)]],
                         tail_v, sem).wait()
        pltpu.sync_copy(tail_v, g_hbm.at[pl.ds(base + NFULL * SB, TAIL)])

    return k(table, idx_kj)


def _k3(sbf, g, W_sbf1, W_sbf2, W_down, W_up):
    T, SR = sbf.shape
    H = g.shape[1]
    BT = 2048
    grid = (T // BT,)

    def body(sbf_r, g_r, W1_r, W2_r, Wd_r, Wu_r, mu_o):
        z = _dot(_dot(sbf_r[...], W1_r[...]), W2_r[...])
        xkd = _silu(_dot(g_r[...], Wd_r[...]))
        mu_o[...] = _dot(xkd * z, Wu_r[...])

    return pl.pallas_call(
        body,
        grid=grid,
        in_specs=[
            pl.BlockSpec((BT, SR), lambda i: (i, 0)),
            pl.BlockSpec((BT, H), lambda i: (i, 0)),
            _full(W_sbf1), _full(W_sbf2), _full(W_down), _full(W_up),
        ],
        out_specs=pl.BlockSpec((BT, H), lambda i: (i, 0)),
        out_shape=jax.ShapeDtypeStruct((T, H), jnp.float32),
    )(sbf, g, W_sbf1, W_sbf2, W_down, W_up)


def _r2a(idx2):
    NR = idx2.shape[0]
    grid = (NR // 8,)

    def body(ix_r, pl_o, c_o):
        d = ix_r[...] >> 13
        rows = []
        for b in range(NBIN):
            rows.append(jnp.where(d == b, 1.0, 0.0))
        OS = jnp.concatenate(rows, axis=0)

        gi = lax.broadcasted_iota(jnp.int32, (NBIN * 8, NBIN * 8), 0)
        gj = lax.broadcasted_iota(jnp.int32, (NBIN * 8, NBIN * 8), 1)
        BD = jnp.where((gi // 8 == gj // 8) & (gj < gi), 1.0, 0.0)
        rowtot = _dot(OS, jnp.ones((128, 1), jnp.float32))
        RP = _dot(BD, rowtot)

        li = lax.broadcasted_iota(jnp.int32, (128, 128), 0)
        lj = lax.broadcasted_iota(jnp.int32, (128, 128), 1)
        U = jnp.where(li < lj, 1.0, 0.0)
        LP = _dot(OS, U)

        pos = jnp.zeros((8, 128), jnp.float32)
        cnt = jnp.zeros((1, 128), jnp.float32)
        for b in range(NBIN):
            Ob = OS[8 * b:8 * b + 8]
            contrib = Ob * (RP[8 * b:8 * b + 8] + LP[8 * b:8 * b + 8])
            pos = pos + contrib
            tot = jnp.sum(Ob)
            oh = jnp.where(
                lax.broadcasted_iota(jnp.int32, (1, 128), 1) == b, 1.0, 0.0)
            cnt = cnt + tot * oh
        pl_o[...] = pos.astype(jnp.int32)
        c_o[...] = cnt[:, :32].reshape(1, 1, 32)

    return pl.pallas_call(
        body,
        grid=grid,
        in_specs=[pl.BlockSpec((8, 128), lambda i: (i, 0))],
        out_specs=[
            pl.BlockSpec((8, 128), lambda i: (i, 0)),
            pl.BlockSpec((1, 1, 32), lambda i: (i, 0, 0)),
        ],
        out_shape=[
            jax.ShapeDtypeStruct((NR, 128), jnp.int32),
            jax.ShapeDtypeStruct((NR // 8, 1, 32), jnp.float32),
        ],
    )(idx2)


def _r2b(C):
    NB = C.shape[0]

    def body(c_r, bm_o, st_o):
        Cv = c_r[...].reshape(NB, 32)
        tot = jnp.sum(Cv, axis=0, keepdims=True)
        bi = lax.broadcasted_iota(jnp.int32, (32, 32), 0)
        bj = lax.broadcasted_iota(jnp.int32, (32, 32), 1)
        U32 = jnp.where(bi < bj, 1.0, 0.0)
        start = _dot(tot, U32)
        ri = lax.broadcasted_iota(jnp.int32, (NB, NB), 0)
        rj = lax.broadcasted_iota(jnp.int32, (NB, NB), 1)
        UB = jnp.where(rj < ri, 1.0, 0.0)
        blkpfx = _dot(UB, Cv)
        bm_o[...] = (blkpfx + start).reshape(NB, 1, 32)
        st_o[...] = start.astype(jnp.int32)

    return pl.pallas_call(
        body,
        in_specs=[_full(C)],
        out_specs=[
            pl.BlockSpec((NB, 1, 32), lambda: (0, 0, 0)),
            pl.BlockSpec((1, 32), lambda: (0, 0)),
        ],
        out_shape=[
            jax.ShapeDtypeStruct((NB, 1, 32), jnp.float32),
            jax.ShapeDtypeStruct((1, 32), jnp.int32),
        ],
    )(C)


def _r2c(idx2, pos_local, Bm):
    NR = idx2.shape[0]
    grid = (NR // 8,)

    def body(ix_r, pl_r, bm_r, pos_o):
        d = ix_r[...] >> 13
        pos = pl_r[...].astype(jnp.float32)
        for b in range(NBIN):
            Ob = jnp.where(d == b, 1.0, 0.0)
            pos = pos + Ob * bm_r[0, 0, b]
        pos_o[...] = pos.astype(jnp.int32)

    return pl.pallas_call(
        body,
        grid=grid,
        in_specs=[
            pl.BlockSpec((8, 128), lambda i: (i, 0)),
            pl.BlockSpec((8, 128), lambda i: (i, 0)),
            pl.BlockSpec((1, 1, 32), lambda i: (i, 0, 0)),
        ],
        out_specs=pl.BlockSpec((8, 128), lambda i: (i, 0)),
        out_shape=jax.ShapeDtypeStruct((NR, 128), jnp.int32),
    )(idx2, pos_local, Bm)


def _k4a(idx2, pos, zeros_i):
    NR = idx2.shape[0]
    T = NR * 128
    NW = NC * NS
    RPW = NR // NW
    REM = NR - RPW * NW
    WPT = T // NS
    ZB = 4000
    mesh = plsc.VectorSubcoreMesh(core_axis_name="c", subcore_axis_name="s")

    @functools.partial(
        pl.kernel,
        mesh=mesh,
        out_type=[
            jax.ShapeDtypeStruct((NC * T + 4096,), jnp.int32),
            jax.ShapeDtypeStruct((NC * T + 4096,), jnp.int32),
        ],
        scratch_types=[
            pltpu.VMEM((1, 128), jnp.int32),
            pltpu.VMEM((1, 128), jnp.int32),
            pltpu.VMEM((1, 128), jnp.int32),
            pltpu.VMEM((ZB,), jnp.int32),
            pltpu.VMEM((ZB,), jnp.int32),
            pltpu.VMEM_SHARED((T,), jnp.int32),
            pltpu.VMEM_SHARED((T,), jnp.int32),
        ],
    )
    def k(ix_hbm, pos_hbm, z_hbm, tid_hbm, dst_hbm,
          posb, valb, tidb, zb, sbuf, tidS, dstS):
        c = lax.axis_index("c")
        s = lax.axis_index("s")
        w = s * NC + c

        pltpu.sync_copy(z_hbm, zb)

        def zero(j, _):
            pltpu.sync_copy(zb, tidS.at[pl.ds(s * WPT + j * ZB, ZB)])
            pltpu.sync_copy(zb, dstS.at[pl.ds(s * WPT + j * ZB, ZB)])
            return 0
        lax.fori_loop(0, WPT // ZB, zero, 0)
        plsc.subcore_barrier()

        def do_row(row, _):
            pltpu.sync_copy(pos_hbm.at[row], posb.at[0])
            pltpu.sync_copy(ix_hbm.at[row], valb.at[0])
            for j in range(8):
                tidb[0, pl.ds(16 * j, 16)] = (
                    row * 128 + 16 * j + lax.iota(jnp.int32, 16))
            pltpu.sync_copy(valb.at[0], dstS.at[posb.at[0]], add=True)
            pltpu.sync_copy(tidb.at[0], tidS.at[posb.at[0]], add=True)
            return 0

        def loop(i, _):
            do_row(w * RPW + i, 0)
            return 0
        lax.fori_loop(0, RPW, loop, 0)

        @pl.when(w < REM)
        def _():
            do_row(NW * RPW + w, 0)

        plsc.subcore_barrier()

        def dump(j, _):
            off = s * WPT + j * ZB
            pltpu.sync_copy(tidS.at[pl.ds(off, ZB)], sbuf)
            pltpu.sync_copy(sbuf, tid_hbm.at[pl.ds(c * T + off, ZB)])
            pltpu.sync_copy(dstS.at[pl.ds(off, ZB)], sbuf)
            pltpu.sync_copy(sbuf, dst_hbm.at[pl.ds(c * T + off, ZB)])
            return 0
        lax.fori_loop(0, WPT // ZB, dump, 0)

    return k(idx2, pos, zeros_i)


def _k4b(mu, tidP, dstP, starti, zeros4):
    T = mu.shape[0]
    H = mu.shape[1]
    ACCR = 8448
    SB = 128
    CH = 16
    CR = CH * SB
    mesh = plsc.VectorSubcoreMesh(core_axis_name="c", subcore_axis_name="s")

    @functools.partial(
        pl.kernel,
        mesh=mesh,
        out_type=jax.ShapeDtypeStruct((NBIN * PRB, H), jnp.float32),
        scratch_types=[
            pltpu.VMEM((1, 32), jnp.int32),
            pltpu.VMEM((CR,), jnp.int32),
            pltpu.VMEM((CR,), jnp.int32),
            pltpu.VMEM((1, CR), jnp.int32),
            pltpu.VMEM((CR,), jnp.int32),
            pltpu.VMEM((CR,), jnp.int32),
            pltpu.VMEM((1, SB), jnp.int32),
            pltpu.VMEM((SB, H), jnp.float32),
            pltpu.VMEM((SB, H), jnp.float32),
            pltpu.VMEM((16, H), jnp.float32),
            pltpu.VMEM_SHARED((ACCR, H), jnp.float32),
            pltpu.SemaphoreType.DMA,
            pltpu.SemaphoreType.DMA,
        ],
    )
    def k(mu_hbm, tid_hbm, dst_hbm, st_hbm, z_hbm, seg_hbm,
          startv, tA, tB, tC, dA, dB, drow, gbuf, gbuf2, zbuf, acc, sem,
          sem2):
        c = lax.axis_index("c")
        s = lax.axis_index("s")
        pltpu.sync_copy(z_hbm, zbuf)
        pltpu.sync_copy(st_hbm, startv)
        v0 = startv[0, pl.ds(0, 16)]
        v1 = startv[0, pl.ds(16, 16)]

        def get_start(kk):
            return v0[kk] if kk < 16 else v1[kk - 16]

        for p in range(NBPC):
            st_a, en_a = get_start(p), get_start(p + 1)
            st_b, en_b = get_start(10 + p), get_start(11 + p)
            st = st_a * (1 - c) + st_b * c
            en = en_a * (1 - c) + en_b * c
            st = jnp.minimum(jnp.maximum(st, 0), T)
            en = jnp.minimum(jnp.maximum(en, st), T)
            lo = (c * NBPC + p) * PRB

            def zero(j, _):
                pltpu.sync_copy(zbuf, acc.at[pl.ds(s * 528 + j * 16, 16)])
                return 0
            lax.fori_loop(0, 33, zero, 0)
            plsc.subcore_barrier()

            b0 = (st >> 7)
            nb = ((en + SB - 1) >> 7) - b0
            span = (nb + NS - 1) // NS
            myb = b0 + s * span
            myn = jnp.minimum(jnp.maximum(nb - s * span, 0), span)
            nch = (myn + CH - 1) // CH

            def chunk(q, _):
                t0c = (myb + q * CH) * SB
                pltpu.sync_copy(tid_hbm.at[pl.ds(t0c, CR)], tA)
                pltpu.sync_copy(tid_hbm.at[pl.ds(T + t0c, CR)], tB)
                pltpu.sync_copy(dst_hbm.at[pl.ds(t0c, CR)], dA)
                pltpu.sync_copy(dst_hbm.at[pl.ds(T + t0c, CR)], dB)
                nin = jnp.minimum(myn - q * CH, CH)

                def mk_tC(j):
                    off = j * SB
                    for m in range(SB // 16):
                        tv = (tA[pl.ds(off + 16 * m, 16)]
                              + tB[pl.ds(off + 16 * m, 16)])
                        tC[0, pl.ds(off + 16 * m, 16)] = jnp.minimum(
                            jnp.maximum(tv, 0), T - 1)
                    return off

                def scat(j, gb):
                    off = j * SB
                    for m in range(SB // 16):
                        v = (dA[pl.ds(off + 16 * m, 16)]
                             + dB[pl.ds(off + 16 * m, 16)])
                        oc = jnp.minimum(jnp.maximum(v - lo, -8), PRB) + 8
                        drow[0, pl.ds(16 * m, 16)] = oc
                    pltpu.sync_copy(gb, acc.at[drow.at[0]], add=True)

                def pair(j2, _):
                    j0 = 2 * j2
                    o0 = mk_tC(j0)
                    d0 = pltpu.async_copy(
                        mu_hbm.at[tC.at[0, pl.ds(o0, SB)]], gbuf, sem)
                    o1 = mk_tC(j0 + 1)
                    d1 = pltpu.async_copy(
                        mu_hbm.at[tC.at[0, pl.ds(o1, SB)]], gbuf2, sem2)
                    d0.wait()
                    scat(j0, gbuf)
                    d1.wait()
                    scat(j0 + 1, gbuf2)
                    return 0
                lax.fori_loop(0, nin >> 1, pair, 0)

                @pl.when((nin & 1) == 1)
                def _():
                    jl = nin - 1
                    ol = mk_tC(jl)
                    pltpu.async_copy(
                        mu_hbm.at[tC.at[0, pl.ds(ol, SB)]],
                        gbuf, sem).wait()
                    scat(jl, gbuf)
                return 0
            lax.fori_loop(0, nch, chunk, 0)
            plsc.subcore_barrier()

            pltpu.sync_copy(acc.at[pl.ds(8 + s * 512, 512)],
                            seg_hbm.at[pl.ds(lo + s * 512, 512)])
            plsc.subcore_barrier()

    return k(mu, tidP, dstP, starti, zeros4)


def _k5(seg, x_ji, x, orbf, bW, bB, W_lin, b_lin, aW, aB):
    E, H = x.shape
    BE = 1600
    grid = (E // BE,)

    def body(seg_r, xji_r, x_r, orbf_r, bW_r, bB_r, Wl_r, bl_r,
             aW_r, aB_r, t_o):
        h = xji_r[...] + _silu(seg_r[...])
        for l in range(bW_r.shape[0]):
            u = _silu(_dot(h, bW_r[l, 0]) + bB_r[l, 0])
            h = h + _silu(_dot(u, bW_r[l, 1]) + bB_r[l, 1])
        h = _silu(_dot(h, Wl_r[...]) + bl_r[...]) + x_r[...]
        for l in range(aW_r.shape[0]):
            u = _silu(_dot(h, aW_r[l, 0]) + aB_r[l, 0])
            h = h + _silu(_dot(u, aW_r[l, 1]) + aB_r[l, 1])
        t_o[...] = orbf_r[...] * h

    return pl.pallas_call(
        body,
        grid=grid,
        in_specs=[
            pl.BlockSpec((BE, H), lambda i: (i, 0)),
            pl.BlockSpec((BE, H), lambda i: (i, 0)),
            pl.BlockSpec((BE, H), lambda i: (i, 0)),
            pl.BlockSpec((BE, H), lambda i: (i, 0)),
            _full(bW), _full(bB), _full(W_lin), _full(b_lin),
            _full(aW), _full(aB),
        ],
        out_specs=pl.BlockSpec((BE, H), lambda i: (i, 0)),
        out_shape=jax.ShapeDtypeStruct((E, H), jnp.float32),
    )(seg, x_ji, x, orbf, bW, bB, W_lin, b_lin, aW, aB)


def _k6_node_scatter(t_arr, idx_i, zeros6):
    E, H = t_arr.shape
    EPC = E // NC
    EPT = EPC // NS
    SB = 128
    NFULL = EPT // SB
    TAIL = EPT - NFULL * SB
    ACCR = 10240
    mesh = plsc.VectorSubcoreMesh(core_axis_name="c", subcore_axis_name="s")

    @functools.partial(
        pl.kernel,
        mesh=mesh,
        out_type=jax.ShapeDtypeStruct((NC, N_NODES, H), jnp.float32),
        scratch_types=[
            pltpu.VMEM((NFULL + 1, SB), jnp.int32),
            pltpu.VMEM((SB, H), jnp.float32),
            pltpu.VMEM((TAIL, H), jnp.float32),
            pltpu.VMEM((16, H), jnp.float32),
            pltpu.VMEM_SHARED((ACCR, H), jnp.float32),
            pltpu.SemaphoreType.DMA,
        ],
    )
    def k(t_hbm, idx_hbm, z_hbm, part_hbm, idx_v, mbuf, tbuf, zbuf, acc, sem):
        c = lax.axis_index("c")
        s = lax.axis_index("s")
        ebase = c * EPC + s * EPT

        def load_idx(b, _):
            pltpu.sync_copy(idx_hbm.at[pl.ds(ebase + b * SB, SB)], idx_v.at[b])
            return 0
        lax.fori_loop(0, NFULL, load_idx, 0)
        pltpu.sync_copy(idx_hbm.at[pl.ds(ebase + NFULL * SB, TAIL)],
                        idx_v.at[NFULL, pl.ds(0, TAIL)])

        pltpu.sync_copy(z_hbm, zbuf)

        def zero(j, _):
            pltpu.sync_copy(zbuf, acc.at[pl.ds(s * 640 + j * 16, 16)])
            return 0
        lax.fori_loop(0, 40, zero, 0)
        plsc.subcore_barrier()

        def scat(b, _):
            pltpu.sync_copy(t_hbm.at[pl.ds(ebase + b * SB, SB)], mbuf)
            pltpu.sync_copy(mbuf, acc.at[idx_v.at[b]], add=True)
            return 0
        lax.fori_loop(0, NFULL, scat, 0)
        pltpu.sync_copy(t_hbm.at[pl.ds(ebase + NFULL * SB, TAIL)], tbuf)
        pltpu.sync_copy(tbuf, acc.at[idx_v.at[NFULL, pl.ds(0, TAIL)]],
                        add=True)
        plsc.subcore_barrier()

        pltpu.sync_copy(acc.at[pl.ds(s * 624, 624)],
                        part_hbm.at[c, pl.ds(s * 624, 624)])

        @pl.when(s == NS - 1)
        def _():
            pltpu.sync_copy(acc.at[pl.ds(9984, 16)],
                            part_hbm.at[c, pl.ds(9984, 16)])

    return k(t_arr, idx_i, zeros6)


def _k7(part, W_oup, b_oup, oW, oB, W_out):
    H = part.shape[2]
    OC = W_out.shape[1]
    BN = 2000
    grid = (N_NODES // BN,)

    def body(p_r, Wo_r, bo_r, oW_r, oB_r, Wout_r, out_o):
        tt = p_r[0] + p_r[1]
        y = _dot(tt, Wo_r[...]) + bo_r[...]
        for l in range(oW_r.shape[0]):
            y = _silu(_dot(y, oW_r[l]) + oB_r[l])
        out_o[...] = _dot(y, Wout_r[...])

    return pl.pallas_call(
        body,
        grid=grid,
        in_specs=[
            pl.BlockSpec((NC, BN, H), lambda i: (0, i, 0)),
            _full(W_oup), _full(b_oup), _full(oW), _full(oB), _full(W_out),
        ],
        out_specs=pl.BlockSpec((BN, OC), lambda i: (i, 0)),
        out_shape=jax.ShapeDtypeStruct((N_NODES, OC), jnp.float32),
    )(part, W_oup, b_oup, oW, oB, W_out)


def kernel(x, rbf, sbf, idx_kj, idx_ji, idx_i,
           W_ji, b_ji, W_kj, b_kj, W_rbf1, W_rbf2, W_sbf1, W_sbf2,
           W_down, W_up, bW, bB, W_lin, b_lin, aW, aB,
           W_orbf, W_oup, b_oup, oW, oB, W_out):
    H = x.shape[1]
    T = idx_ji.shape[0]
    zeros16 = jnp.zeros((16, H), jnp.float32)

    x_ji, xkj_mid, orbf = _k1(x, rbf, W_ji, b_ji, W_kj, b_kj,
                              W_rbf1, W_rbf2, W_orbf)
    g = _k2_gather(xkj_mid, idx_kj)
    mu = _k3(sbf, g, W_sbf1, W_sbf2, W_down, W_up)

    idx2 = idx_ji.reshape(T // 128, 128)
    pos_local, C = _r2a(idx2)
    Bm, starti = _r2b(C)
    pos = _r2c(idx2, pos_local, Bm)
    zeros_i = jnp.zeros((4000,), jnp.int32)
    tidP, dstP = _k4a(idx2, pos, zeros_i)
    seg_pad = _k4b(mu, tidP, dstP, starti, zeros16)

    t_arr = _k5(seg_pad, x_ji, x, orbf, bW, bB, W_lin, b_lin, aW, aB)
    part = _k6_node_scatter(t_arr, idx_i, zeros16)
    return _k7(part, W_oup, b_oup, oW, oB, W_out)
